# Initial kernel scaffold; baseline (speedup 1.0000x reference)
import jax
import jax.numpy as jnp
from jax.experimental import pallas as pl


def kernel(x, edge_index, edge_attr, batch, lin1_w, lin1_b, g_lin1_w, g_lin2_w, g_att_l, g_att_r, g_bias, gru0_wih, gru0_whh, gru0_bih, gru0_bhh, a_lin_w, a_att_src, a_att_dst, a_bias, gru1_wih, gru1_whh, gru1_bih, gru1_bhh, m_lin_w, m_att_src, m_att_dst, m_bias, mgru_wih, mgru_whh, mgru_bih, mgru_bhh, lin2_w, lin2_b):
    raise NotImplementedError("write your pallas kernel here")


# speedup vs baseline: 1.0170x; 1.0170x over previous
import functools

import jax
import jax.numpy as jnp
from jax.experimental import pallas as pl

N = 10000
E = 320000
IN = 128
ED = 16
H = 200
B = 64
NS = 0.01


def _lrelu(v):
    return jnp.where(v > 0, v, NS * v)


def _seg_softmax(a, idx, num):
    m = jax.ops.segment_max(a, idx, num_segments=num)
    m = jnp.where(jnp.isfinite(m), m, 0.0)
    e = jnp.exp(a - m[idx])
    s = jax.ops.segment_sum(e, idx, num_segments=num)
    return e / (s[idx] + 1e-16)


def _gru(inp, h, wih, whh, bih, bhh):
    gi = inp @ wih.T + bih
    gh = h @ whh.T + bhh
    ir, iz, inn = jnp.split(gi, 3, axis=-1)
    hr, hz, hn = jnp.split(gh, 3, axis=-1)
    r = jax.nn.sigmoid(ir + hr)
    z = jax.nn.sigmoid(iz + hz)
    n = jnp.tanh(inn + r * hn)
    return (1.0 - z) * n + z * h


def _lin1_body(x_ref, w_ref, b_ref, o_ref):
    acc = jnp.dot(x_ref[...], w_ref[...], preferred_element_type=jnp.float32)
    acc = acc + b_ref[...]
    o_ref[...] = jnp.where(acc > 0, acc, NS * acc)


@jax.jit
def _lin1(x, w, b):
    blk = 1000
    return pl.pallas_call(
        _lin1_body,
        grid=(N // blk,),
        in_specs=[
            pl.BlockSpec((blk, IN), lambda i: (i, 0)),
            pl.BlockSpec((IN, H), lambda i: (0, 0)),
            pl.BlockSpec((H,), lambda i: (0,)),
        ],
        out_specs=pl.BlockSpec((blk, H), lambda i: (i, 0)),
        out_shape=jax.ShapeDtypeStruct((N, H), jnp.float32),
    )(x, w, b)


def kernel(x, edge_index, edge_attr, batch, lin1_w, lin1_b, g_lin1_w, g_lin2_w, g_att_l, g_att_r, g_bias, gru0_wih, gru0_whh, gru0_bih, gru0_bhh, a_lin_w, a_att_src, a_att_dst, a_bias, gru1_wih, gru1_whh, gru1_bih, gru1_bhh, m_lin_w, m_att_src, m_att_dst, m_bias, mgru_wih, mgru_whh, mgru_bih, mgru_bhh, lin2_w, lin2_b):
    src, dst = edge_index[0], edge_index[1]
    x = _lin1(x, lin1_w, lin1_b)
    xj, xi = x[src], x[dst]
    he = _lrelu(jnp.concatenate([xj, edge_attr], axis=-1) @ g_lin1_w)
    alpha = _lrelu(he @ g_att_l + xi @ g_att_r)
    alpha = _seg_softmax(alpha, dst, N)
    h = jax.ops.segment_sum((xj @ g_lin2_w) * alpha[:, None], dst, num_segments=N) + g_bias
    h = jax.nn.elu(h)
    x = jax.nn.relu(_gru(h, x, gru0_wih, gru0_whh, gru0_bih, gru0_bhh))
    xl = x @ a_lin_w
    alpha = _lrelu((xl @ a_att_src)[src] + (xl @ a_att_dst)[dst])
    alpha = _seg_softmax(alpha, dst, N)
    h = jax.ops.segment_sum(xl[src] * alpha[:, None], dst, num_segments=N) + a_bias
    h = jax.nn.elu(h)
    x = jax.nn.relu(_gru(h, x, gru1_wih, gru1_whh, gru1_bih, gru1_bhh))
    out = jax.nn.relu(jax.ops.segment_sum(x, batch, num_segments=B))
    for _ in range(2):
        xs = x @ m_lin_w
        od = out @ m_lin_w
        alpha = _lrelu(xs @ m_att_src + (od @ m_att_dst)[batch])
        alpha = _seg_softmax(alpha, batch, B)
        h = jax.ops.segment_sum(xs * alpha[:, None], batch, num_segments=B) + m_bias
        h = jax.nn.elu(h)
        out = jax.nn.relu(_gru(h, out, mgru_wih, mgru_whh, mgru_bih, mgru_bhh))
    return (out @ lin2_w + lin2_b).squeeze(-1)


# speedup vs baseline: 2.2197x; 2.1827x over previous
import functools

import jax
import jax.numpy as jnp
from jax import lax
from jax.experimental import pallas as pl
from jax.experimental.pallas import tpu as pltpu
from jax.experimental.pallas import tpu_sc as plsc

N = 10000
E = 320000
IN = 128
ED = 16
H = 200
B = 64
NS = 0.01


def _lrelu(v):
    return jnp.where(v > 0, v, NS * v)


def _seg_softmax(a, idx, num):
    m = jax.ops.segment_max(a, idx, num_segments=num)
    m = jnp.where(jnp.isfinite(m), m, 0.0)
    e = jnp.exp(a - m[idx])
    s = jax.ops.segment_sum(e, idx, num_segments=num)
    return e / (s[idx] + 1e-16)


def _gru(inp, h, wih, whh, bih, bhh):
    gi = inp @ wih.T + bih
    gh = h @ whh.T + bhh
    ir, iz, inn = jnp.split(gi, 3, axis=-1)
    hr, hz, hn = jnp.split(gh, 3, axis=-1)
    r = jax.nn.sigmoid(ir + hr)
    z = jax.nn.sigmoid(iz + hz)
    n = jnp.tanh(inn + r * hn)
    return (1.0 - z) * n + z * h


_NC, _NSUB, _LN = 2, 16, 16
_CH = 80
_EPT = E // _NSUB
_NCHUNK = _EPT // _CH
_NP = 10240
_NPT = _NP // _NSUB
_HH = 128
_HPAD = 2 * _HH

_sc_mesh = plsc.VectorSubcoreMesh(core_axis_name="c", subcore_axis_name="s")


@functools.partial(
    pl.kernel,
    out_type=[
        jax.ShapeDtypeStruct((_NC, _NP, _HH), jnp.float32),
    ],
    mesh=_sc_mesh,
    compiler_params=pltpu.CompilerParams(needs_layout_passes=False),
    scratch_types=[
        pltpu.VMEM((_NP,), jnp.float32),
        pltpu.VMEM((_NP,), jnp.float32),
        pltpu.VMEM((_CH,), jnp.int32),
        pltpu.VMEM((_CH,), jnp.int32),
        pltpu.VMEM((_CH,), jnp.int32),
        pltpu.VMEM((_CH,), jnp.float32),
        pltpu.VMEM((_CH, _HH), jnp.float32),
        pltpu.VMEM_SHARED((_NP, _HH), jnp.float32),
        pltpu.SemaphoreType.DMA,
    ],
)
def _gat_edge_sc(s_hbm, d_hbm, src_hbm, dst_hbm, table_hbm, zeros_hbm,
                 hsum_hbm,
                 s_tab, d_tab, src_v, dst_v, srcg_v, exp_v, rows_v,
                 sh_h, sem):
    c = lax.axis_index("c")
    s = lax.axis_index("s")
    nslice = pl.ds(s * _NPT, _NPT)
    pltpu.sync_copy(zeros_hbm.at[nslice, :], sh_h.at[nslice, :])
    pltpu.sync_copy(s_hbm, s_tab)
    pltpu.sync_copy(d_hbm, d_tab)
    plsc.subcore_barrier()

    cN = c * N

    def chunk(i, _):
        base = s * _EPT + i * _CH
        pltpu.sync_copy(src_hbm.at[pl.ds(base, _CH)], src_v)
        pltpu.sync_copy(dst_hbm.at[pl.ds(base, _CH)], dst_v)

        def grp(g, _):
            sl = pl.ds(g * _LN, _LN)
            sv = src_v[sl]
            dv = dst_v[sl]
            srcg_v[sl] = sv + cN
            a = plsc.load_gather(s_tab, [sv]) + plsc.load_gather(d_tab, [dv])
            a = jnp.maximum(a, NS * a)
            exp_v[sl] = jnp.exp(a)
            return 0
        lax.fori_loop(0, _CH // _LN, grp, 0)

        pltpu.async_copy(table_hbm.at[srcg_v], rows_v, sem).wait()

        def scale(g, _):
            ev = exp_v[pl.ds(g * _LN, _LN)]
            for j in range(_LN):
                wv = jnp.full((_LN,), ev[j], jnp.float32)
                row = g * _LN + j
                for k in range(_HH // _LN):
                    sl = pl.ds(k * _LN, _LN)
                    rows_v[row, sl] = rows_v[row, sl] * wv
            return 0
        lax.fori_loop(0, _CH // _LN, scale, 0)

        pltpu.sync_copy(rows_v, sh_h.at[dst_v], add=True)
        return 0

    lax.fori_loop(0, _NCHUNK, chunk, 0)
    plsc.subcore_barrier()
    pltpu.sync_copy(sh_h.at[nslice, :], hsum_hbm.at[c, nslice, :])


def _gat_edge(s, d, src, dst, xl):
    s = jnp.pad(s, (0, _NP - N))
    d = jnp.pad(d, (0, _NP - N))
    xlp = jnp.pad(xl, ((0, 0), (0, _HPAD - H)))
    ones = jnp.ones((N, 1), jnp.float32)
    half1 = jnp.concatenate([xlp[:, _HH:2 * _HH - 1], ones], axis=1)
    table = jnp.concatenate([xlp[:, :_HH], half1], axis=0)
    zeros = jnp.zeros((_NP, _HH), jnp.float32)
    (hsum,) = _gat_edge_sc(s, d, src, dst, table, zeros)
    h = jnp.concatenate([hsum[0, :N], hsum[1, :N, :H - _HH]], axis=1)
    segsum = hsum[1, :N, _HH - 1]
    return h / (segsum[:, None] + 1e-16)


def _lin1_body(x_ref, w_ref, b_ref, o_ref):
    acc = jnp.dot(x_ref[...], w_ref[...], preferred_element_type=jnp.float32)
    acc = acc + b_ref[...]
    o_ref[...] = jnp.where(acc > 0, acc, NS * acc)


@jax.jit
def _lin1(x, w, b):
    blk = 1000
    return pl.pallas_call(
        _lin1_body,
        grid=(N // blk,),
        in_specs=[
            pl.BlockSpec((blk, IN), lambda i: (i, 0)),
            pl.BlockSpec((IN, H), lambda i: (0, 0)),
            pl.BlockSpec((H,), lambda i: (0,)),
        ],
        out_specs=pl.BlockSpec((blk, H), lambda i: (i, 0)),
        out_shape=jax.ShapeDtypeStruct((N, H), jnp.float32),
    )(x, w, b)


def kernel(x, edge_index, edge_attr, batch, lin1_w, lin1_b, g_lin1_w, g_lin2_w, g_att_l, g_att_r, g_bias, gru0_wih, gru0_whh, gru0_bih, gru0_bhh, a_lin_w, a_att_src, a_att_dst, a_bias, gru1_wih, gru1_whh, gru1_bih, gru1_bhh, m_lin_w, m_att_src, m_att_dst, m_bias, mgru_wih, mgru_whh, mgru_bih, mgru_bhh, lin2_w, lin2_b):
    src, dst = edge_index[0], edge_index[1]
    x = _lin1(x, lin1_w, lin1_b)
    xj, xi = x[src], x[dst]
    he = _lrelu(jnp.concatenate([xj, edge_attr], axis=-1) @ g_lin1_w)
    alpha = _lrelu(he @ g_att_l + xi @ g_att_r)
    alpha = _seg_softmax(alpha, dst, N)
    h = jax.ops.segment_sum((xj @ g_lin2_w) * alpha[:, None], dst, num_segments=N) + g_bias
    h = jax.nn.elu(h)
    x = jax.nn.relu(_gru(h, x, gru0_wih, gru0_whh, gru0_bih, gru0_bhh))
    xl = x @ a_lin_w
    h = _gat_edge(xl @ a_att_src, xl @ a_att_dst, src, dst, xl) + a_bias
    h = jax.nn.elu(h)
    x = jax.nn.relu(_gru(h, x, gru1_wih, gru1_whh, gru1_bih, gru1_bhh))
    out = jax.nn.relu(jax.ops.segment_sum(x, batch, num_segments=B))
    for _ in range(2):
        xs = x @ m_lin_w
        od = out @ m_lin_w
        alpha = _lrelu(xs @ m_att_src + (od @ m_att_dst)[batch])
        alpha = _seg_softmax(alpha, batch, B)
        h = jax.ops.segment_sum(xs * alpha[:, None], batch, num_segments=B) + m_bias
        h = jax.nn.elu(h)
        out = jax.nn.relu(_gru(h, out, mgru_wih, mgru_whh, mgru_bih, mgru_bhh))
    return (out @ lin2_w + lin2_b).squeeze(-1)


# speedup vs baseline: 3.3733x; 1.5197x over previous
import functools

import jax
import jax.numpy as jnp
from jax import lax
from jax.experimental import pallas as pl
from jax.experimental.pallas import tpu as pltpu
from jax.experimental.pallas import tpu_sc as plsc

N = 10000
E = 320000
IN = 128
ED = 16
H = 200
B = 64
NS = 0.01


def _lrelu(v):
    return jnp.where(v > 0, v, NS * v)


def _seg_softmax(a, idx, num):
    m = jax.ops.segment_max(a, idx, num_segments=num)
    m = jnp.where(jnp.isfinite(m), m, 0.0)
    e = jnp.exp(a - m[idx])
    s = jax.ops.segment_sum(e, idx, num_segments=num)
    return e / (s[idx] + 1e-16)


def _gru(inp, h, wih, whh, bih, bhh):
    gi = inp @ wih.T + bih
    gh = h @ whh.T + bhh
    ir, iz, inn = jnp.split(gi, 3, axis=-1)
    hr, hz, hn = jnp.split(gh, 3, axis=-1)
    r = jax.nn.sigmoid(ir + hr)
    z = jax.nn.sigmoid(iz + hz)
    n = jnp.tanh(inn + r * hn)
    return (1.0 - z) * n + z * h


_NC, _NSUB, _LN = 2, 16, 16
_CH = 80
_EPT = E // _NSUB
_NCHUNK = _EPT // _CH
_NP = 10240
_NPT = _NP // _NSUB
_HH = 128
_HPAD = 2 * _HH

_sc_mesh = plsc.VectorSubcoreMesh(core_axis_name="c", subcore_axis_name="s")


@functools.partial(
    pl.kernel,
    out_type=[
        jax.ShapeDtypeStruct((_NC, _NP, _HH), jnp.float32),
    ],
    mesh=_sc_mesh,
    compiler_params=pltpu.CompilerParams(needs_layout_passes=False),
    scratch_types=[
        pltpu.VMEM((_NP,), jnp.float32),
        pltpu.VMEM((_NP,), jnp.float32),
        pltpu.VMEM((_CH,), jnp.int32),
        pltpu.VMEM((_CH,), jnp.int32),
        pltpu.VMEM((_CH,), jnp.int32),
        pltpu.VMEM((_CH,), jnp.float32),
        pltpu.VMEM((_CH, _HH), jnp.float32),
        pltpu.VMEM_SHARED((_NP, _HH), jnp.float32),
        pltpu.SemaphoreType.DMA,
    ],
)
def _gat_edge_sc(s_hbm, d_hbm, src_hbm, dst_hbm, table_hbm, zeros_hbm,
                 hsum_hbm,
                 s_tab, d_tab, src_v, dst_v, srcg_v, exp_v, rows_v,
                 sh_h, sem):
    c = lax.axis_index("c")
    s = lax.axis_index("s")
    nslice = pl.ds(s * _NPT, _NPT)
    pltpu.sync_copy(zeros_hbm.at[nslice, :], sh_h.at[nslice, :])
    pltpu.sync_copy(s_hbm, s_tab)
    pltpu.sync_copy(d_hbm, d_tab)
    plsc.subcore_barrier()

    cN = c * N

    def chunk(i, _):
        base = s * _EPT + i * _CH
        pltpu.sync_copy(src_hbm.at[pl.ds(base, _CH)], src_v)
        pltpu.sync_copy(dst_hbm.at[pl.ds(base, _CH)], dst_v)

        def grp(g, _):
            sl = pl.ds(g * _LN, _LN)
            sv = src_v[sl]
            dv = dst_v[sl]
            srcg_v[sl] = sv + cN
            a = plsc.load_gather(s_tab, [sv]) + plsc.load_gather(d_tab, [dv])
            a = jnp.maximum(a, NS * a)
            exp_v[sl] = jnp.exp(a)
            return 0
        lax.fori_loop(0, _CH // _LN, grp, 0)

        pltpu.async_copy(table_hbm.at[srcg_v], rows_v, sem).wait()

        def scale(g, _):
            ev = exp_v[pl.ds(g * _LN, _LN)]
            for j in range(_LN):
                wv = jnp.full((_LN,), ev[j], jnp.float32)
                row = g * _LN + j
                for k in range(_HH // _LN):
                    sl = pl.ds(k * _LN, _LN)
                    rows_v[row, sl] = rows_v[row, sl] * wv
            return 0
        lax.fori_loop(0, _CH // _LN, scale, 0)

        pltpu.sync_copy(rows_v, sh_h.at[dst_v], add=True)
        return 0

    lax.fori_loop(0, _NCHUNK, chunk, 0)
    plsc.subcore_barrier()
    pltpu.sync_copy(sh_h.at[nslice, :], hsum_hbm.at[c, nslice, :])


def _gat_edge(s, d, src, dst, xl):
    s = jnp.pad(s, (0, _NP - N))
    d = jnp.pad(d, (0, _NP - N))
    xlp = jnp.pad(xl, ((0, 0), (0, _HPAD - H)))
    ones = jnp.ones((N, 1), jnp.float32)
    half1 = jnp.concatenate([xlp[:, _HH:2 * _HH - 1], ones], axis=1)
    table = jnp.concatenate([xlp[:, :_HH], half1], axis=0)
    zeros = jnp.zeros((_NP, _HH), jnp.float32)
    (hsum,) = _gat_edge_sc(s, d, src, dst, table, zeros)
    h = jnp.concatenate([hsum[0, :N], hsum[1, :N, :H - _HH]], axis=1)
    segsum = hsum[1, :N, _HH - 1]
    return h / (segsum[:, None] + 1e-16)


_EPT32 = E // (_NC * _NSUB)
_NCH32 = _EPT32 // _CH
_FW = 2 * _HH


@functools.partial(
    pl.kernel,
    out_type=[jax.ShapeDtypeStruct((E,), jnp.float32)],
    mesh=_sc_mesh,
    compiler_params=pltpu.CompilerParams(needs_layout_passes=False),
    scratch_types=[
        pltpu.VMEM((_FW,), jnp.float32),
        pltpu.VMEM((_NP,), jnp.float32),
        pltpu.VMEM((_CH,), jnp.int32),
        pltpu.VMEM((_CH,), jnp.int32),
        pltpu.VMEM((_CH,), jnp.float32),
        pltpu.VMEM((_CH, _FW), jnp.float32),
        pltpu.VMEM((_CH, _FW), jnp.float32),
        pltpu.SemaphoreType.DMA,
    ],
)
def _gate_logit_sc(nodeA_hbm, eB_hbm, attl_hbm, xr_hbm, src_hbm, dst_hbm,
                   expv_hbm,
                   attl_v, xr_tab, src_v, dst_v, exp_v, rowsA, rowsB, sem):
    c = lax.axis_index("c")
    s = lax.axis_index("s")
    wid = s * _NC + c
    pltpu.sync_copy(attl_hbm, attl_v)
    pltpu.sync_copy(xr_hbm, xr_tab)
    lanes = lax.iota(jnp.int32, _LN)

    def chunk(i, _):
        base = wid * _EPT32 + i * _CH
        pltpu.sync_copy(src_hbm.at[pl.ds(base, _CH)], src_v)
        pltpu.sync_copy(dst_hbm.at[pl.ds(base, _CH)], dst_v)
        pltpu.async_copy(nodeA_hbm.at[src_v], rowsA, sem).wait()
        pltpu.sync_copy(eB_hbm.at[pl.ds(base, _CH), :], rowsB)

        def grp(g, _):
            sl = pl.ds(g * _LN, _LN)
            rows16 = g * _LN + lanes

            def fchunk(fc, acc):
                attw16 = attl_v[pl.ds(fc * _LN, _LN)]
                for jj in range(_LN):
                    colf = jnp.full((_LN,), fc * _LN + jj, jnp.int32)
                    av = plsc.load_gather(rowsA, [rows16, colf])
                    bv = plsc.load_gather(rowsB, [rows16, colf])
                    v = av + bv
                    v = jnp.maximum(v, NS * v)
                    acc = acc + v * jnp.full((_LN,), attw16[jj], jnp.float32)
                return acc

            acc = lax.fori_loop(0, _FW // _LN, fchunk,
                                jnp.zeros((_LN,), jnp.float32))
            lg = acc + plsc.load_gather(xr_tab, [dst_v[sl]])
            lg = jnp.maximum(lg, NS * lg)
            exp_v[sl] = jnp.exp(lg)
            return 0

        lax.fori_loop(0, _CH // _LN, grp, 0)
        pltpu.sync_copy(exp_v, expv_hbm.at[pl.ds(base, _CH)])
        return 0

    lax.fori_loop(0, _NCH32, chunk, 0)


@functools.partial(
    pl.kernel,
    out_type=[jax.ShapeDtypeStruct((_NC, _NP, _HH), jnp.float32)],
    mesh=_sc_mesh,
    compiler_params=pltpu.CompilerParams(needs_layout_passes=False),
    scratch_types=[
        pltpu.VMEM((_CH,), jnp.int32),
        pltpu.VMEM((_CH,), jnp.int32),
        pltpu.VMEM((_CH,), jnp.int32),
        pltpu.VMEM((_CH,), jnp.float32),
        pltpu.VMEM((_CH, _HH), jnp.float32),
        pltpu.VMEM_SHARED((_NP, _HH), jnp.float32),
        pltpu.SemaphoreType.DMA,
    ],
)
def _wscatter_sc(src_hbm, dst_hbm, expv_hbm, table_hbm, zeros_hbm,
                 hsum_hbm,
                 src_v, dst_v, srcg_v, exp_v, rows_v, sh_h, sem):
    c = lax.axis_index("c")
    s = lax.axis_index("s")
    nslice = pl.ds(s * _NPT, _NPT)
    pltpu.sync_copy(zeros_hbm.at[nslice, :], sh_h.at[nslice, :])
    plsc.subcore_barrier()
    cN = c * N

    def chunk(i, _):
        base = s * _EPT + i * _CH
        pltpu.sync_copy(src_hbm.at[pl.ds(base, _CH)], src_v)
        pltpu.sync_copy(dst_hbm.at[pl.ds(base, _CH)], dst_v)
        pltpu.sync_copy(expv_hbm.at[pl.ds(base, _CH)], exp_v)

        def addoff(g, _):
            sl = pl.ds(g * _LN, _LN)
            srcg_v[sl] = src_v[sl] + cN
            return 0
        lax.fori_loop(0, _CH // _LN, addoff, 0)

        pltpu.async_copy(table_hbm.at[srcg_v], rows_v, sem).wait()

        def scale(g, _):
            ev = exp_v[pl.ds(g * _LN, _LN)]
            for j in range(_LN):
                wv = jnp.full((_LN,), ev[j], jnp.float32)
                row = g * _LN + j
                for k in range(_HH // _LN):
                    sl = pl.ds(k * _LN, _LN)
                    rows_v[row, sl] = rows_v[row, sl] * wv
            return 0
        lax.fori_loop(0, _CH // _LN, scale, 0)

        pltpu.sync_copy(rows_v, sh_h.at[dst_v], add=True)
        return 0

    lax.fori_loop(0, _NCHUNK, chunk, 0)
    plsc.subcore_barrier()
    pltpu.sync_copy(sh_h.at[nslice, :], hsum_hbm.at[c, nslice, :])


def _mk_table(xv):
    xp = jnp.pad(xv, ((0, 0), (0, _HPAD - H)))
    ones = jnp.ones((N, 1), jnp.float32)
    half1 = jnp.concatenate([xp[:, _HH:2 * _HH - 1], ones], axis=1)
    return jnp.concatenate([xp[:, :_HH], half1], axis=0)


def _norm_h(hsum):
    h = jnp.concatenate([hsum[0, :N], hsum[1, :N, :H - _HH]], axis=1)
    segsum = hsum[1, :N, _HH - 1]
    return h / (segsum[:, None] + 1e-16)


def _gate_edge(nodeA, eB, attl, xr, src, dst, xw2):
    nodeA = jnp.pad(nodeA, ((0, 0), (0, _FW - H)))
    eB = jnp.pad(eB, ((0, 0), (0, _FW - H)))
    attl = jnp.pad(attl, (0, _FW - H))
    xr = jnp.pad(xr, (0, _NP - N))
    (expv,) = _gate_logit_sc(nodeA, eB, attl, xr, src, dst)
    zeros = jnp.zeros((_NP, _HH), jnp.float32)
    (hsum,) = _wscatter_sc(src, dst, expv, _mk_table(xw2), zeros)
    return _norm_h(hsum)


def _lin1_body(x_ref, w_ref, b_ref, o_ref):
    acc = jnp.dot(x_ref[...], w_ref[...], preferred_element_type=jnp.float32)
    acc = acc + b_ref[...]
    o_ref[...] = jnp.where(acc > 0, acc, NS * acc)


@jax.jit
def _lin1(x, w, b):
    blk = 1000
    return pl.pallas_call(
        _lin1_body,
        grid=(N // blk,),
        in_specs=[
            pl.BlockSpec((blk, IN), lambda i: (i, 0)),
            pl.BlockSpec((IN, H), lambda i: (0, 0)),
            pl.BlockSpec((H,), lambda i: (0,)),
        ],
        out_specs=pl.BlockSpec((blk, H), lambda i: (i, 0)),
        out_shape=jax.ShapeDtypeStruct((N, H), jnp.float32),
    )(x, w, b)


def kernel(x, edge_index, edge_attr, batch, lin1_w, lin1_b, g_lin1_w, g_lin2_w, g_att_l, g_att_r, g_bias, gru0_wih, gru0_whh, gru0_bih, gru0_bhh, a_lin_w, a_att_src, a_att_dst, a_bias, gru1_wih, gru1_whh, gru1_bih, gru1_bhh, m_lin_w, m_att_src, m_att_dst, m_bias, mgru_wih, mgru_whh, mgru_bih, mgru_bhh, lin2_w, lin2_b):
    src, dst = edge_index[0], edge_index[1]
    x = _lin1(x, lin1_w, lin1_b)
    nodeA = x @ g_lin1_w[:H]
    eB = edge_attr @ g_lin1_w[H:]
    xr = x @ g_att_r
    h = _gate_edge(nodeA, eB, g_att_l, xr, src, dst, x @ g_lin2_w) + g_bias
    h = jax.nn.elu(h)
    x = jax.nn.relu(_gru(h, x, gru0_wih, gru0_whh, gru0_bih, gru0_bhh))
    xl = x @ a_lin_w
    h = _gat_edge(xl @ a_att_src, xl @ a_att_dst, src, dst, xl) + a_bias
    h = jax.nn.elu(h)
    x = jax.nn.relu(_gru(h, x, gru1_wih, gru1_whh, gru1_bih, gru1_bhh))
    out = jax.nn.relu(jax.ops.segment_sum(x, batch, num_segments=B))
    for _ in range(2):
        xs = x @ m_lin_w
        od = out @ m_lin_w
        alpha = _lrelu(xs @ m_att_src + (od @ m_att_dst)[batch])
        alpha = _seg_softmax(alpha, batch, B)
        h = jax.ops.segment_sum(xs * alpha[:, None], batch, num_segments=B) + m_bias
        h = jax.nn.elu(h)
        out = jax.nn.relu(_gru(h, out, mgru_wih, mgru_whh, mgru_bih, mgru_bhh))
    return (out @ lin2_w + lin2_b).squeeze(-1)


# speedup vs baseline: 5.3198x; 1.5770x over previous
import functools

import jax
import jax.numpy as jnp
from jax import lax
from jax.experimental import pallas as pl
from jax.experimental.pallas import tpu as pltpu
from jax.experimental.pallas import tpu_sc as plsc

N = 10000
E = 320000
IN = 128
ED = 16
H = 200
B = 64
NS = 0.01


def _lrelu(v):
    return jnp.where(v > 0, v, NS * v)


def _seg_softmax(a, idx, num):
    m = jax.ops.segment_max(a, idx, num_segments=num)
    m = jnp.where(jnp.isfinite(m), m, 0.0)
    e = jnp.exp(a - m[idx])
    s = jax.ops.segment_sum(e, idx, num_segments=num)
    return e / (s[idx] + 1e-16)


def _gru(inp, h, wih, whh, bih, bhh):
    gi = inp @ wih.T + bih
    gh = h @ whh.T + bhh
    ir, iz, inn = jnp.split(gi, 3, axis=-1)
    hr, hz, hn = jnp.split(gh, 3, axis=-1)
    r = jax.nn.sigmoid(ir + hr)
    z = jax.nn.sigmoid(iz + hz)
    n = jnp.tanh(inn + r * hn)
    return (1.0 - z) * n + z * h


_NC, _NSUB, _LN = 2, 16, 16
_CH = 80
_EPT = E // _NSUB
_NCHUNK = _EPT // _CH
_NP = 10240
_NPT = _NP // _NSUB
_HH = 128
_HPAD = 2 * _HH

_sc_mesh = plsc.VectorSubcoreMesh(core_axis_name="c", subcore_axis_name="s")


@functools.partial(
    pl.kernel,
    out_type=[
        jax.ShapeDtypeStruct((_NC, _NP, _HH), jnp.float32),
    ],
    mesh=_sc_mesh,
    compiler_params=pltpu.CompilerParams(needs_layout_passes=False),
    scratch_types=[
        pltpu.VMEM((_NP,), jnp.float32),
        pltpu.VMEM((_NP,), jnp.float32),
        pltpu.VMEM((_CH,), jnp.int32),
        pltpu.VMEM((_CH,), jnp.int32),
        pltpu.VMEM((_CH,), jnp.int32),
        pltpu.VMEM((_CH,), jnp.float32),
        pltpu.VMEM((_CH, _HH), jnp.float32),
        pltpu.VMEM_SHARED((_NP, _HH), jnp.float32),
        pltpu.SemaphoreType.DMA,
    ],
)
def _gat_edge_sc(s_hbm, d_hbm, src_hbm, dst_hbm, table_hbm, zeros_hbm,
                 hsum_hbm,
                 s_tab, d_tab, src_v, dst_v, srcg_v, exp_v, rows_v,
                 sh_h, sem):
    c = lax.axis_index("c")
    s = lax.axis_index("s")
    nslice = pl.ds(s * _NPT, _NPT)
    pltpu.sync_copy(zeros_hbm.at[nslice, :], sh_h.at[nslice, :])
    pltpu.sync_copy(s_hbm, s_tab)
    pltpu.sync_copy(d_hbm, d_tab)
    plsc.subcore_barrier()

    cN = c * N

    def chunk(i, _):
        base = s * _EPT + i * _CH
        pltpu.sync_copy(src_hbm.at[pl.ds(base, _CH)], src_v)
        pltpu.sync_copy(dst_hbm.at[pl.ds(base, _CH)], dst_v)

        def grp(g, _):
            sl = pl.ds(g * _LN, _LN)
            sv = src_v[sl]
            dv = dst_v[sl]
            srcg_v[sl] = sv + cN
            a = plsc.load_gather(s_tab, [sv]) + plsc.load_gather(d_tab, [dv])
            a = jnp.maximum(a, NS * a)
            exp_v[sl] = jnp.exp(a)
            return 0
        lax.fori_loop(0, _CH // _LN, grp, 0)

        pltpu.async_copy(table_hbm.at[srcg_v], rows_v, sem).wait()

        def scale(g, _):
            ev = exp_v[pl.ds(g * _LN, _LN)]
            for j in range(_LN):
                wv = jnp.full((_LN,), ev[j], jnp.float32)
                row = g * _LN + j
                for k in range(_HH // _LN):
                    sl = pl.ds(k * _LN, _LN)
                    rows_v[row, sl] = rows_v[row, sl] * wv
            return 0
        lax.fori_loop(0, _CH // _LN, scale, 0)

        pltpu.sync_copy(rows_v, sh_h.at[dst_v], add=True)
        return 0

    lax.fori_loop(0, _NCHUNK, chunk, 0)
    plsc.subcore_barrier()
    pltpu.sync_copy(sh_h.at[nslice, :], hsum_hbm.at[c, nslice, :])


def _gat_edge(s, d, src, dst, xl):
    s = jnp.pad(s, (0, _NP - N))
    d = jnp.pad(d, (0, _NP - N))
    xlp = jnp.pad(xl, ((0, 0), (0, _HPAD - H)))
    ones = jnp.ones((N, 1), jnp.float32)
    half1 = jnp.concatenate([xlp[:, _HH:2 * _HH - 1], ones], axis=1)
    table = jnp.concatenate([xlp[:, :_HH], half1], axis=0)
    zeros = jnp.zeros((_NP, _HH), jnp.float32)
    (hsum,) = _gat_edge_sc(s, d, src, dst, table, zeros)
    h = jnp.concatenate([hsum[0, :N], hsum[1, :N, :H - _HH]], axis=1)
    segsum = hsum[1, :N, _HH - 1]
    return h / (segsum[:, None] + 1e-16)


_EPT32 = E // (_NC * _NSUB)
_NCH32 = _EPT32 // _CH
_FW = 2 * _HH


@functools.partial(
    pl.kernel,
    out_type=[jax.ShapeDtypeStruct((E,), jnp.float32)],
    mesh=_sc_mesh,
    compiler_params=pltpu.CompilerParams(needs_layout_passes=False),
    scratch_types=[
        pltpu.VMEM((_FW,), jnp.float32),
        pltpu.VMEM((_NP,), jnp.float32),
        pltpu.VMEM((_CH,), jnp.int32),
        pltpu.VMEM((_CH,), jnp.int32),
        pltpu.VMEM((_CH,), jnp.float32),
        pltpu.VMEM((_CH, _FW), jnp.float32),
        pltpu.VMEM((_CH, _FW), jnp.float32),
        pltpu.VMEM((_LN, _LN), jnp.float32),
        pltpu.SemaphoreType.DMA,
    ],
)
def _gate_logit_sc(nodeA_hbm, eB_hbm, attl_hbm, xr_hbm, src_hbm, dst_hbm,
                   expv_hbm,
                   attl_v, xr_tab, src_v, dst_v, exp_v, rowsA, rowsB, accbuf, sem):
    c = lax.axis_index("c")
    s = lax.axis_index("s")
    wid = s * _NC + c
    pltpu.sync_copy(attl_hbm, attl_v)
    pltpu.sync_copy(xr_hbm, xr_tab)
    lanes = lax.iota(jnp.int32, _LN)

    def chunk(i, _):
        base = wid * _EPT32 + i * _CH
        pltpu.sync_copy(src_hbm.at[pl.ds(base, _CH)], src_v)
        pltpu.sync_copy(dst_hbm.at[pl.ds(base, _CH)], dst_v)
        pltpu.async_copy(nodeA_hbm.at[src_v], rowsA, sem).wait()
        pltpu.sync_copy(eB_hbm.at[pl.ds(base, _CH), :], rowsB)

        def grp(g, _):
            sl = pl.ds(g * _LN, _LN)

            def edge(j, _):
                row = g * _LN + j
                acc = jnp.zeros((_LN,), jnp.float32)
                for fc in range(_FW // _LN):
                    sl2 = pl.ds(fc * _LN, _LN)
                    v = rowsA[row, sl2] + rowsB[row, sl2]
                    v = jnp.maximum(v, NS * v)
                    acc = acc + v * attl_v[sl2]
                accbuf[j, :] = acc
                return 0
            lax.fori_loop(0, _LN, edge, 0)

            def tcol(jc, tot):
                colv = jnp.full((_LN,), jc, jnp.int32)
                return tot + plsc.load_gather(accbuf, [lanes, colv])
            tot = lax.fori_loop(0, _LN, tcol, jnp.zeros((_LN,), jnp.float32))
            lg = tot + plsc.load_gather(xr_tab, [dst_v[sl]])
            lg = jnp.maximum(lg, NS * lg)
            exp_v[sl] = jnp.exp(lg)
            return 0

        lax.fori_loop(0, _CH // _LN, grp, 0)
        pltpu.sync_copy(exp_v, expv_hbm.at[pl.ds(base, _CH)])
        return 0

    lax.fori_loop(0, _NCH32, chunk, 0)


@functools.partial(
    pl.kernel,
    out_type=[jax.ShapeDtypeStruct((_NC, _NP, _HH), jnp.float32)],
    mesh=_sc_mesh,
    compiler_params=pltpu.CompilerParams(needs_layout_passes=False),
    scratch_types=[
        pltpu.VMEM((_CH,), jnp.int32),
        pltpu.VMEM((_CH,), jnp.int32),
        pltpu.VMEM((_CH,), jnp.int32),
        pltpu.VMEM((_CH,), jnp.float32),
        pltpu.VMEM((_CH, _HH), jnp.float32),
        pltpu.VMEM_SHARED((_NP, _HH), jnp.float32),
        pltpu.SemaphoreType.DMA,
    ],
)
def _wscatter_sc(src_hbm, dst_hbm, expv_hbm, table_hbm, zeros_hbm,
                 hsum_hbm,
                 src_v, dst_v, srcg_v, exp_v, rows_v, sh_h, sem):
    c = lax.axis_index("c")
    s = lax.axis_index("s")
    nslice = pl.ds(s * _NPT, _NPT)
    pltpu.sync_copy(zeros_hbm.at[nslice, :], sh_h.at[nslice, :])
    plsc.subcore_barrier()
    cN = c * N

    def chunk(i, _):
        base = s * _EPT + i * _CH
        pltpu.sync_copy(src_hbm.at[pl.ds(base, _CH)], src_v)
        pltpu.sync_copy(dst_hbm.at[pl.ds(base, _CH)], dst_v)
        pltpu.sync_copy(expv_hbm.at[pl.ds(base, _CH)], exp_v)

        def addoff(g, _):
            sl = pl.ds(g * _LN, _LN)
            srcg_v[sl] = src_v[sl] + cN
            return 0
        lax.fori_loop(0, _CH // _LN, addoff, 0)

        pltpu.async_copy(table_hbm.at[srcg_v], rows_v, sem).wait()

        def scale(g, _):
            ev = exp_v[pl.ds(g * _LN, _LN)]
            for j in range(_LN):
                wv = jnp.full((_LN,), ev[j], jnp.float32)
                row = g * _LN + j
                for k in range(_HH // _LN):
                    sl = pl.ds(k * _LN, _LN)
                    rows_v[row, sl] = rows_v[row, sl] * wv
            return 0
        lax.fori_loop(0, _CH // _LN, scale, 0)

        pltpu.sync_copy(rows_v, sh_h.at[dst_v], add=True)
        return 0

    lax.fori_loop(0, _NCHUNK, chunk, 0)
    plsc.subcore_barrier()
    pltpu.sync_copy(sh_h.at[nslice, :], hsum_hbm.at[c, nslice, :])


def _mk_table(xv):
    xp = jnp.pad(xv, ((0, 0), (0, _HPAD - H)))
    ones = jnp.ones((N, 1), jnp.float32)
    half1 = jnp.concatenate([xp[:, _HH:2 * _HH - 1], ones], axis=1)
    return jnp.concatenate([xp[:, :_HH], half1], axis=0)


def _norm_h(hsum):
    h = jnp.concatenate([hsum[0, :N], hsum[1, :N, :H - _HH]], axis=1)
    segsum = hsum[1, :N, _HH - 1]
    return h / (segsum[:, None] + 1e-16)


def _gate_edge(nodeA, eB, attl, xr, src, dst, xw2):
    nodeA = jnp.pad(nodeA, ((0, 0), (0, _FW - H)))
    eB = jnp.pad(eB, ((0, 0), (0, _FW - H)))
    attl = jnp.pad(attl, (0, _FW - H))
    xr = jnp.pad(xr, (0, _NP - N))
    (expv,) = _gate_logit_sc(nodeA, eB, attl, xr, src, dst)
    zeros = jnp.zeros((_NP, _HH), jnp.float32)
    (hsum,) = _wscatter_sc(src, dst, expv, _mk_table(xw2), zeros)
    return _norm_h(hsum)


def _lin1_body(x_ref, w_ref, b_ref, o_ref):
    acc = jnp.dot(x_ref[...], w_ref[...], preferred_element_type=jnp.float32)
    acc = acc + b_ref[...]
    o_ref[...] = jnp.where(acc > 0, acc, NS * acc)


@jax.jit
def _lin1(x, w, b):
    blk = 1000
    return pl.pallas_call(
        _lin1_body,
        grid=(N // blk,),
        in_specs=[
            pl.BlockSpec((blk, IN), lambda i: (i, 0)),
            pl.BlockSpec((IN, H), lambda i: (0, 0)),
            pl.BlockSpec((H,), lambda i: (0,)),
        ],
        out_specs=pl.BlockSpec((blk, H), lambda i: (i, 0)),
        out_shape=jax.ShapeDtypeStruct((N, H), jnp.float32),
    )(x, w, b)


def kernel(x, edge_index, edge_attr, batch, lin1_w, lin1_b, g_lin1_w, g_lin2_w, g_att_l, g_att_r, g_bias, gru0_wih, gru0_whh, gru0_bih, gru0_bhh, a_lin_w, a_att_src, a_att_dst, a_bias, gru1_wih, gru1_whh, gru1_bih, gru1_bhh, m_lin_w, m_att_src, m_att_dst, m_bias, mgru_wih, mgru_whh, mgru_bih, mgru_bhh, lin2_w, lin2_b):
    src, dst = edge_index[0], edge_index[1]
    x = _lin1(x, lin1_w, lin1_b)
    nodeA = x @ g_lin1_w[:H]
    eB = edge_attr @ g_lin1_w[H:]
    xr = x @ g_att_r
    h = _gate_edge(nodeA, eB, g_att_l, xr, src, dst, x @ g_lin2_w) + g_bias
    h = jax.nn.elu(h)
    x = jax.nn.relu(_gru(h, x, gru0_wih, gru0_whh, gru0_bih, gru0_bhh))
    xl = x @ a_lin_w
    h = _gat_edge(xl @ a_att_src, xl @ a_att_dst, src, dst, xl) + a_bias
    h = jax.nn.elu(h)
    x = jax.nn.relu(_gru(h, x, gru1_wih, gru1_whh, gru1_bih, gru1_bhh))
    out = jax.nn.relu(jax.ops.segment_sum(x, batch, num_segments=B))
    for _ in range(2):
        xs = x @ m_lin_w
        od = out @ m_lin_w
        alpha = _lrelu(xs @ m_att_src + (od @ m_att_dst)[batch])
        alpha = _seg_softmax(alpha, batch, B)
        h = jax.ops.segment_sum(xs * alpha[:, None], batch, num_segments=B) + m_bias
        h = jax.nn.elu(h)
        out = jax.nn.relu(_gru(h, out, mgru_wih, mgru_whh, mgru_bih, mgru_bhh))
    return (out @ lin2_w + lin2_b).squeeze(-1)


# speedup vs baseline: 5.5687x; 1.0468x over previous
import functools

import jax
import jax.numpy as jnp
from jax import lax
from jax.experimental import pallas as pl
from jax.experimental.pallas import tpu as pltpu
from jax.experimental.pallas import tpu_sc as plsc

N = 10000
E = 320000
IN = 128
ED = 16
H = 200
B = 64
NS = 0.01


def _lrelu(v):
    return jnp.where(v > 0, v, NS * v)


def _seg_softmax(a, idx, num):
    m = jax.ops.segment_max(a, idx, num_segments=num)
    m = jnp.where(jnp.isfinite(m), m, 0.0)
    e = jnp.exp(a - m[idx])
    s = jax.ops.segment_sum(e, idx, num_segments=num)
    return e / (s[idx] + 1e-16)


def _gru(inp, h, wih, whh, bih, bhh):
    gi = inp @ wih.T + bih
    gh = h @ whh.T + bhh
    ir, iz, inn = jnp.split(gi, 3, axis=-1)
    hr, hz, hn = jnp.split(gh, 3, axis=-1)
    r = jax.nn.sigmoid(ir + hr)
    z = jax.nn.sigmoid(iz + hz)
    n = jnp.tanh(inn + r * hn)
    return (1.0 - z) * n + z * h


_NC, _NSUB, _LN = 2, 16, 16
_CH = 80
_EPT = E // _NSUB
_NCHUNK = _EPT // _CH
_NP = 10240
_NPT = _NP // _NSUB
_HH = 128
_HPAD = 2 * _HH

_sc_mesh = plsc.VectorSubcoreMesh(core_axis_name="c", subcore_axis_name="s")


@functools.partial(
    pl.kernel,
    out_type=[
        jax.ShapeDtypeStruct((_NC, _NP, _HH), jnp.float32),
    ],
    mesh=_sc_mesh,
    compiler_params=pltpu.CompilerParams(needs_layout_passes=False),
    scratch_types=[
        pltpu.VMEM((_NP,), jnp.float32),
        pltpu.VMEM((_NP,), jnp.float32),
        pltpu.VMEM((_CH,), jnp.int32),
        pltpu.VMEM((_CH,), jnp.int32),
        pltpu.VMEM((_CH,), jnp.int32),
        pltpu.VMEM((_CH,), jnp.float32),
        pltpu.VMEM((_CH, _HH), jnp.float32),
        pltpu.VMEM_SHARED((_NP, _HH), jnp.float32),
        pltpu.SemaphoreType.DMA,
    ],
)
def _gat_edge_sc(s_hbm, d_hbm, src_hbm, dst_hbm, table_hbm, zeros_hbm,
                 hsum_hbm,
                 s_tab, d_tab, src_v, dst_v, srcg_v, exp_v, rows_v,
                 sh_h, sem):
    c = lax.axis_index("c")
    s = lax.axis_index("s")
    nslice = pl.ds(s * _NPT, _NPT)
    pltpu.sync_copy(zeros_hbm.at[nslice, :], sh_h.at[nslice, :])
    pltpu.sync_copy(s_hbm, s_tab)
    pltpu.sync_copy(d_hbm, d_tab)
    plsc.subcore_barrier()

    cN = c * N

    def chunk(i, _):
        base = s * _EPT + i * _CH
        pltpu.sync_copy(src_hbm.at[pl.ds(base, _CH)], src_v)
        pltpu.sync_copy(dst_hbm.at[pl.ds(base, _CH)], dst_v)

        def grp(g, _):
            sl = pl.ds(g * _LN, _LN)
            sv = src_v[sl]
            dv = dst_v[sl]
            srcg_v[sl] = sv + cN
            a = plsc.load_gather(s_tab, [sv]) + plsc.load_gather(d_tab, [dv])
            a = jnp.maximum(a, NS * a)
            exp_v[sl] = jnp.exp(a)
            return 0
        lax.fori_loop(0, _CH // _LN, grp, 0)

        pltpu.async_copy(table_hbm.at[srcg_v], rows_v, sem).wait()

        def scale(g, _):
            for j in range(_LN):
                row = g * _LN + j
                wv = plsc.load_gather(exp_v, [jnp.full((_LN,), row, jnp.int32)])
                for k in range(_HH // _LN):
                    sl = pl.ds(k * _LN, _LN)
                    rows_v[row, sl] = rows_v[row, sl] * wv
            return 0
        lax.fori_loop(0, _CH // _LN, scale, 0)

        pltpu.sync_copy(rows_v, sh_h.at[dst_v], add=True)
        return 0

    lax.fori_loop(0, _NCHUNK, chunk, 0)
    plsc.subcore_barrier()
    pltpu.sync_copy(sh_h.at[nslice, :], hsum_hbm.at[c, nslice, :])


def _gat_edge(s, d, src, dst, xl):
    s = jnp.pad(s, (0, _NP - N))
    d = jnp.pad(d, (0, _NP - N))
    xlp = jnp.pad(xl, ((0, 0), (0, _HPAD - H)))
    ones = jnp.ones((N, 1), jnp.float32)
    half1 = jnp.concatenate([xlp[:, _HH:2 * _HH - 1], ones], axis=1)
    table = jnp.concatenate([xlp[:, :_HH], half1], axis=0)
    zeros = jnp.zeros((_NP, _HH), jnp.float32)
    (hsum,) = _gat_edge_sc(s, d, src, dst, table, zeros)
    h = jnp.concatenate([hsum[0, :N], hsum[1, :N, :H - _HH]], axis=1)
    segsum = hsum[1, :N, _HH - 1]
    return h / (segsum[:, None] + 1e-16)


_EPT32 = E // (_NC * _NSUB)
_NCH32 = _EPT32 // _CH
_FW = 2 * _HH


@functools.partial(
    pl.kernel,
    out_type=[jax.ShapeDtypeStruct((E,), jnp.float32)],
    mesh=_sc_mesh,
    compiler_params=pltpu.CompilerParams(needs_layout_passes=False),
    scratch_types=[
        pltpu.VMEM((_FW,), jnp.float32),
        pltpu.VMEM((_NP,), jnp.float32),
        pltpu.VMEM((_CH,), jnp.int32),
        pltpu.VMEM((_CH,), jnp.int32),
        pltpu.VMEM((_CH,), jnp.int32),
        pltpu.VMEM((_CH,), jnp.int32),
        pltpu.VMEM((_CH,), jnp.float32),
        pltpu.VMEM((_CH, _FW), jnp.float32),
        pltpu.VMEM((_CH, _FW), jnp.float32),
        pltpu.VMEM((_CH, _FW), jnp.float32),
        pltpu.VMEM((_CH, _FW), jnp.float32),
        pltpu.VMEM((_LN, _LN), jnp.float32),
        pltpu.SemaphoreType.DMA,
        pltpu.SemaphoreType.DMA,
        pltpu.SemaphoreType.DMA,
        pltpu.SemaphoreType.DMA,
    ],
)
def _gate_logit_sc(nodeA_hbm, eB_hbm, attl_hbm, xr_hbm, src_hbm, dst_hbm,
                   expv_hbm,
                   attl_v, xr_tab, src0, src1, dst0, dst1, exp_v,
                   rowsA0, rowsA1, rowsB0, rowsB1, accbuf,
                   semA0, semA1, semB0, semB1):
    c = lax.axis_index("c")
    s = lax.axis_index("s")
    wid = s * _NC + c
    pltpu.sync_copy(attl_hbm, attl_v)
    pltpu.sync_copy(xr_hbm, xr_tab)
    lanes = lax.iota(jnp.int32, _LN)
    bufs = ((src0, dst0, rowsA0, rowsB0, semA0, semB0),
            (src1, dst1, rowsA1, rowsB1, semA1, semB1))

    def start(i, b):
        srcb, dstb, rA, rB, sA, sB = bufs[b]
        base = wid * _EPT32 + i * _CH
        pltpu.sync_copy(src_hbm.at[pl.ds(base, _CH)], srcb)
        pltpu.sync_copy(dst_hbm.at[pl.ds(base, _CH)], dstb)
        pltpu.async_copy(nodeA_hbm.at[srcb], rA, sA)
        pltpu.async_copy(eB_hbm.at[pl.ds(base, _CH), :], rB, sB)

    def compute(i, b):
        srcb, dstb, rA, rB, sA, sB = bufs[b]
        base = wid * _EPT32 + i * _CH
        pltpu.make_async_copy(nodeA_hbm.at[srcb], rA, sA).wait()
        pltpu.make_async_copy(eB_hbm.at[pl.ds(base, _CH), :], rB, sB).wait()

        def grp(g, _):
            sl = pl.ds(g * _LN, _LN)

            def edge(j, _):
                row = g * _LN + j
                acc = jnp.zeros((_LN,), jnp.float32)
                for fc in range(_FW // _LN):
                    sl2 = pl.ds(fc * _LN, _LN)
                    v = rA[row, sl2] + rB[row, sl2]
                    v = jnp.maximum(v, NS * v)
                    acc = acc + v * attl_v[sl2]
                accbuf[j, :] = acc
                return 0
            lax.fori_loop(0, _LN, edge, 0)

            def tcol(jc, tot):
                colv = jnp.full((_LN,), jc, jnp.int32)
                return tot + plsc.load_gather(accbuf, [lanes, colv])
            tot = lax.fori_loop(0, _LN, tcol, jnp.zeros((_LN,), jnp.float32))
            lg = tot + plsc.load_gather(xr_tab, [dstb[sl]])
            lg = jnp.maximum(lg, NS * lg)
            exp_v[sl] = jnp.exp(lg)
            return 0

        lax.fori_loop(0, _CH // _LN, grp, 0)
        pltpu.sync_copy(exp_v, expv_hbm.at[pl.ds(base, _CH)])

    npairs = (_NCH32 - 1) // 2
    start(0, 0)

    def pair(p, _):
        i0 = 2 * p
        start(i0 + 1, 1)
        compute(i0, 0)

        @pl.when(p < npairs - 1)
        def _():
            start(i0 + 2, 0)
        compute(i0 + 1, 1)
        return 0

    lax.fori_loop(0, npairs, pair, 0)
    start(_NCH32 - 1, 0)
    compute(_NCH32 - 1, 0)


@functools.partial(
    pl.kernel,
    out_type=[jax.ShapeDtypeStruct((_NC, _NP, _HH), jnp.float32)],
    mesh=_sc_mesh,
    compiler_params=pltpu.CompilerParams(needs_layout_passes=False),
    scratch_types=[
        pltpu.VMEM((_CH,), jnp.int32),
        pltpu.VMEM((_CH,), jnp.int32),
        pltpu.VMEM((_CH,), jnp.int32),
        pltpu.VMEM((_CH,), jnp.float32),
        pltpu.VMEM((_CH, _HH), jnp.float32),
        pltpu.VMEM_SHARED((_NP, _HH), jnp.float32),
        pltpu.SemaphoreType.DMA,
    ],
)
def _wscatter_sc(src_hbm, dst_hbm, expv_hbm, table_hbm, zeros_hbm,
                 hsum_hbm,
                 src_v, dst_v, srcg_v, exp_v, rows_v, sh_h, sem):
    c = lax.axis_index("c")
    s = lax.axis_index("s")
    nslice = pl.ds(s * _NPT, _NPT)
    pltpu.sync_copy(zeros_hbm.at[nslice, :], sh_h.at[nslice, :])
    plsc.subcore_barrier()
    cN = c * N

    def chunk(i, _):
        base = s * _EPT + i * _CH
        pltpu.sync_copy(src_hbm.at[pl.ds(base, _CH)], src_v)
        pltpu.sync_copy(dst_hbm.at[pl.ds(base, _CH)], dst_v)
        pltpu.sync_copy(expv_hbm.at[pl.ds(base, _CH)], exp_v)

        def addoff(g, _):
            sl = pl.ds(g * _LN, _LN)
            srcg_v[sl] = src_v[sl] + cN
            return 0
        lax.fori_loop(0, _CH // _LN, addoff, 0)

        pltpu.async_copy(table_hbm.at[srcg_v], rows_v, sem).wait()

        def scale(g, _):
            for j in range(_LN):
                row = g * _LN + j
                wv = plsc.load_gather(exp_v, [jnp.full((_LN,), row, jnp.int32)])
                for k in range(_HH // _LN):
                    sl = pl.ds(k * _LN, _LN)
                    rows_v[row, sl] = rows_v[row, sl] * wv
            return 0
        lax.fori_loop(0, _CH // _LN, scale, 0)

        pltpu.sync_copy(rows_v, sh_h.at[dst_v], add=True)
        return 0

    lax.fori_loop(0, _NCHUNK, chunk, 0)
    plsc.subcore_barrier()
    pltpu.sync_copy(sh_h.at[nslice, :], hsum_hbm.at[c, nslice, :])


def _mk_table(xv):
    xp = jnp.pad(xv, ((0, 0), (0, _HPAD - H)))
    ones = jnp.ones((N, 1), jnp.float32)
    half1 = jnp.concatenate([xp[:, _HH:2 * _HH - 1], ones], axis=1)
    return jnp.concatenate([xp[:, :_HH], half1], axis=0)


def _norm_h(hsum):
    h = jnp.concatenate([hsum[0, :N], hsum[1, :N, :H - _HH]], axis=1)
    segsum = hsum[1, :N, _HH - 1]
    return h / (segsum[:, None] + 1e-16)


def _gate_edge(nodeA, eB, attl, xr, src, dst, xw2):
    nodeA = jnp.pad(nodeA, ((0, 0), (0, _FW - H)))
    eB = jnp.pad(eB, ((0, 0), (0, _FW - H)))
    attl = jnp.pad(attl, (0, _FW - H))
    xr = jnp.pad(xr, (0, _NP - N))
    (expv,) = _gate_logit_sc(nodeA, eB, attl, xr, src, dst)
    zeros = jnp.zeros((_NP, _HH), jnp.float32)
    (hsum,) = _wscatter_sc(src, dst, expv, _mk_table(xw2), zeros)
    return _norm_h(hsum)


def _lin1_body(x_ref, w_ref, b_ref, o_ref):
    acc = jnp.dot(x_ref[...], w_ref[...], preferred_element_type=jnp.float32)
    acc = acc + b_ref[...]
    o_ref[...] = jnp.where(acc > 0, acc, NS * acc)


@jax.jit
def _lin1(x, w, b):
    blk = 1000
    return pl.pallas_call(
        _lin1_body,
        grid=(N // blk,),
        in_specs=[
            pl.BlockSpec((blk, IN), lambda i: (i, 0)),
            pl.BlockSpec((IN, H), lambda i: (0, 0)),
            pl.BlockSpec((H,), lambda i: (0,)),
        ],
        out_specs=pl.BlockSpec((blk, H), lambda i: (i, 0)),
        out_shape=jax.ShapeDtypeStruct((N, H), jnp.float32),
    )(x, w, b)


def kernel(x, edge_index, edge_attr, batch, lin1_w, lin1_b, g_lin1_w, g_lin2_w, g_att_l, g_att_r, g_bias, gru0_wih, gru0_whh, gru0_bih, gru0_bhh, a_lin_w, a_att_src, a_att_dst, a_bias, gru1_wih, gru1_whh, gru1_bih, gru1_bhh, m_lin_w, m_att_src, m_att_dst, m_bias, mgru_wih, mgru_whh, mgru_bih, mgru_bhh, lin2_w, lin2_b):
    src, dst = edge_index[0], edge_index[1]
    x = _lin1(x, lin1_w, lin1_b)
    nodeA = x @ g_lin1_w[:H]
    eB = edge_attr @ g_lin1_w[H:]
    xr = x @ g_att_r
    h = _gate_edge(nodeA, eB, g_att_l, xr, src, dst, x @ g_lin2_w) + g_bias
    h = jax.nn.elu(h)
    x = jax.nn.relu(_gru(h, x, gru0_wih, gru0_whh, gru0_bih, gru0_bhh))
    xl = x @ a_lin_w
    h = _gat_edge(xl @ a_att_src, xl @ a_att_dst, src, dst, xl) + a_bias
    h = jax.nn.elu(h)
    x = jax.nn.relu(_gru(h, x, gru1_wih, gru1_whh, gru1_bih, gru1_bhh))
    out = jax.nn.relu(jax.ops.segment_sum(x, batch, num_segments=B))
    for _ in range(2):
        xs = x @ m_lin_w
        od = out @ m_lin_w
        alpha = _lrelu(xs @ m_att_src + (od @ m_att_dst)[batch])
        alpha = _seg_softmax(alpha, batch, B)
        h = jax.ops.segment_sum(xs * alpha[:, None], batch, num_segments=B) + m_bias
        h = jax.nn.elu(h)
        out = jax.nn.relu(_gru(h, out, mgru_wih, mgru_whh, mgru_bih, mgru_bhh))
    return (out @ lin2_w + lin2_b).squeeze(-1)


# speedup vs baseline: 6.8714x; 1.2339x over previous
import functools

import jax
import jax.numpy as jnp
from jax import lax
from jax.experimental import pallas as pl
from jax.experimental.pallas import tpu as pltpu
from jax.experimental.pallas import tpu_sc as plsc

N = 10000
E = 320000
IN = 128
ED = 16
H = 200
B = 64
NS = 0.01


def _lrelu(v):
    return jnp.where(v > 0, v, NS * v)


def _seg_softmax(a, idx, num):
    m = jax.ops.segment_max(a, idx, num_segments=num)
    m = jnp.where(jnp.isfinite(m), m, 0.0)
    e = jnp.exp(a - m[idx])
    s = jax.ops.segment_sum(e, idx, num_segments=num)
    return e / (s[idx] + 1e-16)


def _gru(inp, h, wih, whh, bih, bhh):
    gi = inp @ wih.T + bih
    gh = h @ whh.T + bhh
    ir, iz, inn = jnp.split(gi, 3, axis=-1)
    hr, hz, hn = jnp.split(gh, 3, axis=-1)
    r = jax.nn.sigmoid(ir + hr)
    z = jax.nn.sigmoid(iz + hz)
    n = jnp.tanh(inn + r * hn)
    return (1.0 - z) * n + z * h


_NC, _NSUB, _LN = 2, 16, 16
_CH = 80
_EPT = E // _NSUB
_NCHUNK = _EPT // _CH
_NP = 10240
_NPT = _NP // _NSUB
_HH = 128
_HPAD = 2 * _HH

_sc_mesh = plsc.VectorSubcoreMesh(core_axis_name="c", subcore_axis_name="s")


@functools.partial(
    pl.kernel,
    out_type=[
        jax.ShapeDtypeStruct((_NC, _NP, _HH), jnp.float32),
    ],
    mesh=_sc_mesh,
    compiler_params=pltpu.CompilerParams(needs_layout_passes=False),
    scratch_types=[
        pltpu.VMEM((_NP,), jnp.float32),
        pltpu.VMEM((_NP,), jnp.float32),
        pltpu.VMEM((_CH,), jnp.int32),
        pltpu.VMEM((_CH,), jnp.int32),
        pltpu.VMEM((_CH,), jnp.int32),
        pltpu.VMEM((_CH,), jnp.int32),
        pltpu.VMEM((_CH,), jnp.float32),
        pltpu.VMEM((_CH,), jnp.float32),
        pltpu.VMEM((_CH, _HH), jnp.float32),
        pltpu.VMEM((_CH, _HH), jnp.float32),
        pltpu.VMEM_SHARED((_NP, _HH), jnp.float32),
        pltpu.SemaphoreType.DMA,
        pltpu.SemaphoreType.DMA,
    ],
)
def _gat_edge_sc(s_hbm, d_hbm, src_hbm, dst_hbm, table_hbm, zeros_hbm,
                 hsum_hbm,
                 s_tab, d_tab, srcg0, srcg1, dst0, dst1, exp0, exp1,
                 rows0, rows1, sh_h, semA0, semA1):
    c = lax.axis_index("c")
    s = lax.axis_index("s")
    nslice = pl.ds(s * _NPT, _NPT)
    pltpu.sync_copy(zeros_hbm.at[nslice, :], sh_h.at[nslice, :])
    pltpu.sync_copy(s_hbm, s_tab)
    pltpu.sync_copy(d_hbm, d_tab)
    plsc.subcore_barrier()
    cN = c * N
    bufs = ((srcg0, dst0, exp0, rows0, semA0),
            (srcg1, dst1, exp1, rows1, semA1))

    def start(i, b):
        srcgb, dstb, expb, rowsb, semb = bufs[b]
        base = s * _EPT + i * _CH
        pltpu.sync_copy(src_hbm.at[pl.ds(base, _CH)], srcgb)
        pltpu.sync_copy(dst_hbm.at[pl.ds(base, _CH)], dstb)

        def grp(g, _):
            sl = pl.ds(g * _LN, _LN)
            sv = srcgb[sl]
            dv = dstb[sl]
            srcgb[sl] = sv + cN
            a = plsc.load_gather(s_tab, [sv]) + plsc.load_gather(d_tab, [dv])
            a = jnp.maximum(a, NS * a)
            expb[sl] = jnp.exp(a)
            return 0
        lax.fori_loop(0, _CH // _LN, grp, 0)
        pltpu.async_copy(table_hbm.at[srcgb], rowsb, semb)

    def compute(b):
        srcgb, dstb, expb, rowsb, semb = bufs[b]
        pltpu.make_async_copy(table_hbm.at[srcgb], rowsb, semb).wait()

        def scale(g, _):
            ev = expb[pl.ds(g * _LN, _LN)]
            for j in range(_LN):
                wv = jnp.full((_LN,), ev[j], jnp.float32)
                row = g * _LN + j
                for k in range(_HH // _LN):
                    sl = pl.ds(k * _LN, _LN)
                    rowsb[row, sl] = rowsb[row, sl] * wv
            return 0
        lax.fori_loop(0, _CH // _LN, scale, 0)
        pltpu.sync_copy(rowsb, sh_h.at[dstb], add=True)

    start(0, 0)

    def pair(p, _):
        start(2 * p + 1, 1)
        compute(0)

        @pl.when(p < _NCHUNK // 2 - 1)
        def _():
            start(2 * p + 2, 0)
        compute(1)
        return 0

    lax.fori_loop(0, _NCHUNK // 2, pair, 0)
    plsc.subcore_barrier()
    pltpu.sync_copy(sh_h.at[nslice, :], hsum_hbm.at[c, nslice, :])


def _gat_edge(s, d, src, dst, xl):
    s = jnp.pad(s, (0, _NP - N))
    d = jnp.pad(d, (0, _NP - N))
    xlp = jnp.pad(xl, ((0, 0), (0, _HPAD - H)))
    ones = jnp.ones((N, 1), jnp.float32)
    half1 = jnp.concatenate([xlp[:, _HH:2 * _HH - 1], ones], axis=1)
    table = jnp.concatenate([xlp[:, :_HH], half1], axis=0)
    zeros = jnp.zeros((_NP, _HH), jnp.float32)
    (hsum,) = _gat_edge_sc(s, d, src, dst, table, zeros)
    h = jnp.concatenate([hsum[0, :N], hsum[1, :N, :H - _HH]], axis=1)
    segsum = hsum[1, :N, _HH - 1]
    return h / (segsum[:, None] + 1e-16)


_EPT32 = E // (_NC * _NSUB)
_NCH32 = _EPT32 // _CH
_FW = 2 * _HH


@functools.partial(
    pl.kernel,
    out_type=[jax.ShapeDtypeStruct((E,), jnp.float32)],
    mesh=_sc_mesh,
    compiler_params=pltpu.CompilerParams(needs_layout_passes=False),
    scratch_types=[
        pltpu.VMEM((_FW,), jnp.float32),
        pltpu.VMEM((_NP,), jnp.float32),
        pltpu.VMEM((_CH,), jnp.int32),
        pltpu.VMEM((_CH,), jnp.int32),
        pltpu.VMEM((_CH,), jnp.int32),
        pltpu.VMEM((_CH,), jnp.int32),
        pltpu.VMEM((_CH,), jnp.float32),
        pltpu.VMEM((_CH, _FW), jnp.float32),
        pltpu.VMEM((_CH, _FW), jnp.float32),
        pltpu.VMEM((_CH, _FW), jnp.float32),
        pltpu.VMEM((_CH, _FW), jnp.float32),
        pltpu.VMEM((_LN, _LN), jnp.float32),
        pltpu.SemaphoreType.DMA,
        pltpu.SemaphoreType.DMA,
        pltpu.SemaphoreType.DMA,
        pltpu.SemaphoreType.DMA,
    ],
)
def _gate_logit_sc(nodeA_hbm, eB_hbm, attl_hbm, xr_hbm, src_hbm, dst_hbm,
                   expv_hbm,
                   attl_v, xr_tab, src0, src1, dst0, dst1, exp_v,
                   rowsA0, rowsA1, rowsB0, rowsB1, accbuf,
                   semA0, semA1, semB0, semB1):
    c = lax.axis_index("c")
    s = lax.axis_index("s")
    wid = s * _NC + c
    pltpu.sync_copy(attl_hbm, attl_v)
    pltpu.sync_copy(xr_hbm, xr_tab)
    lanes = lax.iota(jnp.int32, _LN)
    bufs = ((src0, dst0, rowsA0, rowsB0, semA0, semB0),
            (src1, dst1, rowsA1, rowsB1, semA1, semB1))

    def start(i, b):
        srcb, dstb, rA, rB, sA, sB = bufs[b]
        base = wid * _EPT32 + i * _CH
        pltpu.sync_copy(src_hbm.at[pl.ds(base, _CH)], srcb)
        pltpu.sync_copy(dst_hbm.at[pl.ds(base, _CH)], dstb)
        pltpu.async_copy(nodeA_hbm.at[srcb], rA, sA)
        pltpu.async_copy(eB_hbm.at[pl.ds(base, _CH), :], rB, sB)

    def compute(i, b):
        srcb, dstb, rA, rB, sA, sB = bufs[b]
        base = wid * _EPT32 + i * _CH
        pltpu.make_async_copy(nodeA_hbm.at[srcb], rA, sA).wait()
        pltpu.make_async_copy(eB_hbm.at[pl.ds(base, _CH), :], rB, sB).wait()

        def grp(g, _):
            sl = pl.ds(g * _LN, _LN)

            def edge(j, _):
                row = g * _LN + j
                acc = jnp.zeros((_LN,), jnp.float32)
                for fc in range(_FW // _LN):
                    sl2 = pl.ds(fc * _LN, _LN)
                    v = rA[row, sl2] + rB[row, sl2]
                    v = jnp.maximum(v, NS * v)
                    acc = acc + v * attl_v[sl2]
                accbuf[j, :] = acc
                return 0
            lax.fori_loop(0, _LN, edge, 0)

            def tcol(jc, tot):
                colv = jnp.full((_LN,), jc, jnp.int32)
                return tot + plsc.load_gather(accbuf, [lanes, colv])
            tot = lax.fori_loop(0, _LN, tcol, jnp.zeros((_LN,), jnp.float32))
            lg = tot + plsc.load_gather(xr_tab, [dstb[sl]])
            lg = jnp.maximum(lg, NS * lg)
            exp_v[sl] = jnp.exp(lg)
            return 0

        lax.fori_loop(0, _CH // _LN, grp, 0)
        pltpu.sync_copy(exp_v, expv_hbm.at[pl.ds(base, _CH)])

    npairs = (_NCH32 - 1) // 2
    start(0, 0)

    def pair(p, _):
        i0 = 2 * p
        start(i0 + 1, 1)
        compute(i0, 0)

        @pl.when(p < npairs - 1)
        def _():
            start(i0 + 2, 0)
        compute(i0 + 1, 1)
        return 0

    lax.fori_loop(0, npairs, pair, 0)
    start(_NCH32 - 1, 0)
    compute(_NCH32 - 1, 0)


@functools.partial(
    pl.kernel,
    out_type=[jax.ShapeDtypeStruct((_NC, _NP, _HH), jnp.float32)],
    mesh=_sc_mesh,
    compiler_params=pltpu.CompilerParams(needs_layout_passes=False),
    scratch_types=[
        pltpu.VMEM((_CH,), jnp.int32),
        pltpu.VMEM((_CH,), jnp.int32),
        pltpu.VMEM((_CH,), jnp.int32),
        pltpu.VMEM((_CH,), jnp.int32),
        pltpu.VMEM((_CH,), jnp.float32),
        pltpu.VMEM((_CH,), jnp.float32),
        pltpu.VMEM((_CH, _HH), jnp.float32),
        pltpu.VMEM((_CH, _HH), jnp.float32),
        pltpu.VMEM_SHARED((_NP, _HH), jnp.float32),
        pltpu.SemaphoreType.DMA,
        pltpu.SemaphoreType.DMA,
    ],
)
def _wscatter_sc(src_hbm, dst_hbm, expv_hbm, table_hbm, zeros_hbm,
                 hsum_hbm,
                 srcg0, srcg1, dst0, dst1, exp0, exp1, rows0, rows1,
                 sh_h, semA0, semA1):
    c = lax.axis_index("c")
    s = lax.axis_index("s")
    nslice = pl.ds(s * _NPT, _NPT)
    pltpu.sync_copy(zeros_hbm.at[nslice, :], sh_h.at[nslice, :])
    plsc.subcore_barrier()
    cN = c * N
    bufs = ((srcg0, dst0, exp0, rows0, semA0),
            (srcg1, dst1, exp1, rows1, semA1))

    def start(i, b):
        srcgb, dstb, expb, rowsb, semb = bufs[b]
        base = s * _EPT + i * _CH
        pltpu.sync_copy(src_hbm.at[pl.ds(base, _CH)], srcgb)
        pltpu.sync_copy(dst_hbm.at[pl.ds(base, _CH)], dstb)
        pltpu.sync_copy(expv_hbm.at[pl.ds(base, _CH)], expb)

        def addoff(g, _):
            sl = pl.ds(g * _LN, _LN)
            srcgb[sl] = srcgb[sl] + cN
            return 0
        lax.fori_loop(0, _CH // _LN, addoff, 0)
        pltpu.async_copy(table_hbm.at[srcgb], rowsb, semb)

    def compute(b):
        srcgb, dstb, expb, rowsb, semb = bufs[b]
        pltpu.make_async_copy(table_hbm.at[srcgb], rowsb, semb).wait()

        def scale(g, _):
            ev = expb[pl.ds(g * _LN, _LN)]
            for j in range(_LN):
                wv = jnp.full((_LN,), ev[j], jnp.float32)
                row = g * _LN + j
                for k in range(_HH // _LN):
                    sl = pl.ds(k * _LN, _LN)
                    rowsb[row, sl] = rowsb[row, sl] * wv
            return 0
        lax.fori_loop(0, _CH // _LN, scale, 0)
        pltpu.sync_copy(rowsb, sh_h.at[dstb], add=True)

    start(0, 0)

    def pair(p, _):
        start(2 * p + 1, 1)
        compute(0)

        @pl.when(p < _NCHUNK // 2 - 1)
        def _():
            start(2 * p + 2, 0)
        compute(1)
        return 0

    lax.fori_loop(0, _NCHUNK // 2, pair, 0)
    plsc.subcore_barrier()
    pltpu.sync_copy(sh_h.at[nslice, :], hsum_hbm.at[c, nslice, :])


def _mk_table(xv):
    xp = jnp.pad(xv, ((0, 0), (0, _HPAD - H)))
    ones = jnp.ones((N, 1), jnp.float32)
    half1 = jnp.concatenate([xp[:, _HH:2 * _HH - 1], ones], axis=1)
    return jnp.concatenate([xp[:, :_HH], half1], axis=0)


def _norm_h(hsum):
    h = jnp.concatenate([hsum[0, :N], hsum[1, :N, :H - _HH]], axis=1)
    segsum = hsum[1, :N, _HH - 1]
    return h / (segsum[:, None] + 1e-16)


def _gate_edge(nodeA, eB, attl, xr, src, dst, xw2):
    nodeA = jnp.pad(nodeA, ((0, 0), (0, _FW - H)))
    eB = jnp.pad(eB, ((0, 0), (0, _FW - H)))
    attl = jnp.pad(attl, (0, _FW - H))
    xr = jnp.pad(xr, (0, _NP - N))
    (expv,) = _gate_logit_sc(nodeA, eB, attl, xr, src, dst)
    zeros = jnp.zeros((_NP, _HH), jnp.float32)
    (hsum,) = _wscatter_sc(src, dst, expv, _mk_table(xw2), zeros)
    return _norm_h(hsum)


def _lin1_body(x_ref, w_ref, b_ref, o_ref):
    acc = jnp.dot(x_ref[...], w_ref[...], preferred_element_type=jnp.float32)
    acc = acc + b_ref[...]
    o_ref[...] = jnp.where(acc > 0, acc, NS * acc)


@jax.jit
def _lin1(x, w, b):
    blk = 1000
    return pl.pallas_call(
        _lin1_body,
        grid=(N // blk,),
        in_specs=[
            pl.BlockSpec((blk, IN), lambda i: (i, 0)),
            pl.BlockSpec((IN, H), lambda i: (0, 0)),
            pl.BlockSpec((H,), lambda i: (0,)),
        ],
        out_specs=pl.BlockSpec((blk, H), lambda i: (i, 0)),
        out_shape=jax.ShapeDtypeStruct((N, H), jnp.float32),
    )(x, w, b)


def kernel(x, edge_index, edge_attr, batch, lin1_w, lin1_b, g_lin1_w, g_lin2_w, g_att_l, g_att_r, g_bias, gru0_wih, gru0_whh, gru0_bih, gru0_bhh, a_lin_w, a_att_src, a_att_dst, a_bias, gru1_wih, gru1_whh, gru1_bih, gru1_bhh, m_lin_w, m_att_src, m_att_dst, m_bias, mgru_wih, mgru_whh, mgru_bih, mgru_bhh, lin2_w, lin2_b):
    src, dst = edge_index[0], edge_index[1]
    x = _lin1(x, lin1_w, lin1_b)
    nodeA = x @ g_lin1_w[:H]
    eB = edge_attr @ g_lin1_w[H:]
    xr = x @ g_att_r
    h = _gate_edge(nodeA, eB, g_att_l, xr, src, dst, x @ g_lin2_w) + g_bias
    h = jax.nn.elu(h)
    x = jax.nn.relu(_gru(h, x, gru0_wih, gru0_whh, gru0_bih, gru0_bhh))
    xl = x @ a_lin_w
    h = _gat_edge(xl @ a_att_src, xl @ a_att_dst, src, dst, xl) + a_bias
    h = jax.nn.elu(h)
    x = jax.nn.relu(_gru(h, x, gru1_wih, gru1_whh, gru1_bih, gru1_bhh))
    out = jax.nn.relu(jax.ops.segment_sum(x, batch, num_segments=B))
    for _ in range(2):
        xs = x @ m_lin_w
        od = out @ m_lin_w
        alpha = _lrelu(xs @ m_att_src + (od @ m_att_dst)[batch])
        alpha = _seg_softmax(alpha, batch, B)
        h = jax.ops.segment_sum(xs * alpha[:, None], batch, num_segments=B) + m_bias
        h = jax.nn.elu(h)
        out = jax.nn.relu(_gru(h, out, mgru_wih, mgru_whh, mgru_bih, mgru_bhh))
    return (out @ lin2_w + lin2_b).squeeze(-1)


# speedup vs baseline: 9.2637x; 1.3481x over previous
import functools

import jax
import jax.numpy as jnp
from jax import lax
from jax.experimental import pallas as pl
from jax.experimental.pallas import tpu as pltpu
from jax.experimental.pallas import tpu_sc as plsc

N = 10000
E = 320000
IN = 128
ED = 16
H = 200
B = 64
NS = 0.01


def _lrelu(v):
    return jnp.where(v > 0, v, NS * v)


def _seg_softmax(a, idx, num):
    m = jax.ops.segment_max(a, idx, num_segments=num)
    m = jnp.where(jnp.isfinite(m), m, 0.0)
    e = jnp.exp(a - m[idx])
    s = jax.ops.segment_sum(e, idx, num_segments=num)
    return e / (s[idx] + 1e-16)


def _gru(inp, h, wih, whh, bih, bhh):
    gi = inp @ wih.T + bih
    gh = h @ whh.T + bhh
    ir, iz, inn = jnp.split(gi, 3, axis=-1)
    hr, hz, hn = jnp.split(gh, 3, axis=-1)
    r = jax.nn.sigmoid(ir + hr)
    z = jax.nn.sigmoid(iz + hz)
    n = jnp.tanh(inn + r * hn)
    return (1.0 - z) * n + z * h


_NC, _NSUB, _LN = 2, 16, 16
_CH = 80
_EPT = E // _NSUB
_NCHUNK = _EPT // _CH
_NP = 10240
_NPT = _NP // _NSUB
_HH = 128
_HPAD = 2 * _HH

_sc_mesh = plsc.VectorSubcoreMesh(core_axis_name="c", subcore_axis_name="s")


@functools.partial(
    pl.kernel,
    out_type=[
        jax.ShapeDtypeStruct((_NC, _NP, _HH), jnp.float32),
    ],
    mesh=_sc_mesh,
    compiler_params=pltpu.CompilerParams(needs_layout_passes=False),
    scratch_types=[
        pltpu.VMEM((_NP,), jnp.float32),
        pltpu.VMEM((_NP,), jnp.float32),
        pltpu.VMEM((_CH,), jnp.int32),
        pltpu.VMEM((_CH,), jnp.int32),
        pltpu.VMEM((_CH,), jnp.int32),
        pltpu.VMEM((_CH,), jnp.int32),
        pltpu.VMEM((_CH,), jnp.float32),
        pltpu.VMEM((_CH,), jnp.float32),
        pltpu.VMEM((_CH, _HH), jnp.float32),
        pltpu.VMEM((_CH, _HH), jnp.float32),
        pltpu.VMEM_SHARED((_NP, _HH), jnp.float32),
        pltpu.SemaphoreType.DMA,
        pltpu.SemaphoreType.DMA,
    ],
)
def _gat_edge_sc(s_hbm, d_hbm, src_hbm, dst_hbm, table_hbm, zeros_hbm,
                 hsum_hbm,
                 s_tab, d_tab, srcg0, srcg1, dst0, dst1, exp0, exp1,
                 rows0, rows1, sh_h, semA0, semA1):
    c = lax.axis_index("c")
    s = lax.axis_index("s")
    nslice = pl.ds(s * _NPT, _NPT)
    pltpu.sync_copy(zeros_hbm.at[nslice, :], sh_h.at[nslice, :])
    pltpu.sync_copy(s_hbm, s_tab)
    pltpu.sync_copy(d_hbm, d_tab)
    plsc.subcore_barrier()
    cN = c * N
    bufs = ((srcg0, dst0, exp0, rows0, semA0),
            (srcg1, dst1, exp1, rows1, semA1))

    def start(i, b):
        srcgb, dstb, expb, rowsb, semb = bufs[b]
        base = s * _EPT + i * _CH
        pltpu.sync_copy(src_hbm.at[pl.ds(base, _CH)], srcgb)
        pltpu.sync_copy(dst_hbm.at[pl.ds(base, _CH)], dstb)

        def grp(g, _):
            sl = pl.ds(g * _LN, _LN)
            sv = srcgb[sl]
            dv = dstb[sl]
            srcgb[sl] = sv + cN
            a = plsc.load_gather(s_tab, [sv]) + plsc.load_gather(d_tab, [dv])
            a = jnp.maximum(a, NS * a)
            expb[sl] = jnp.exp(a)
            return 0
        lax.fori_loop(0, _CH // _LN, grp, 0)
        pltpu.async_copy(table_hbm.at[srcgb], rowsb, semb)

    def compute(b):
        srcgb, dstb, expb, rowsb, semb = bufs[b]
        pltpu.make_async_copy(table_hbm.at[srcgb], rowsb, semb).wait()

        def scale(g, _):
            ev = expb[pl.ds(g * _LN, _LN)]
            for j in range(_LN):
                wv = jnp.full((_LN,), ev[j], jnp.float32)
                row = g * _LN + j
                for k in range(_HH // _LN):
                    sl = pl.ds(k * _LN, _LN)
                    rowsb[row, sl] = rowsb[row, sl] * wv
            return 0
        lax.fori_loop(0, _CH // _LN, scale, 0)
        pltpu.sync_copy(rowsb, sh_h.at[dstb], add=True)

    start(0, 0)

    def pair(p, _):
        start(2 * p + 1, 1)
        compute(0)

        @pl.when(p < _NCHUNK // 2 - 1)
        def _():
            start(2 * p + 2, 0)
        compute(1)
        return 0

    lax.fori_loop(0, _NCHUNK // 2, pair, 0)
    plsc.subcore_barrier()
    pltpu.sync_copy(sh_h.at[nslice, :], hsum_hbm.at[c, nslice, :])


def _gat_edge(s, d, src, dst, xl):
    s = jnp.pad(s, (0, _NP - N))
    d = jnp.pad(d, (0, _NP - N))
    xlp = jnp.pad(xl, ((0, 0), (0, _HPAD - H)))
    ones = jnp.ones((N, 1), jnp.float32)
    half1 = jnp.concatenate([xlp[:, _HH:2 * _HH - 1], ones], axis=1)
    table = jnp.concatenate([xlp[:, :_HH], half1], axis=0)
    zeros = jnp.zeros((_NP, _HH), jnp.float32)
    (hsum,) = _gat_edge_sc(s, d, src, dst, table, zeros)
    h = jnp.concatenate([hsum[0, :N], hsum[1, :N, :H - _HH]], axis=1)
    segsum = hsum[1, :N, _HH - 1]
    return h / (segsum[:, None] + 1e-16)


_EPT32 = E // (_NC * _NSUB)
_NCH32 = _EPT32 // _CH
_FW = 2 * _HH


@functools.partial(
    pl.kernel,
    out_type=[jax.ShapeDtypeStruct((E,), jnp.float32)],
    mesh=_sc_mesh,
    compiler_params=pltpu.CompilerParams(needs_layout_passes=False),
    scratch_types=[
        pltpu.VMEM((_FW,), jnp.float32),
        pltpu.VMEM((_NP,), jnp.float32),
        pltpu.VMEM((_CH,), jnp.int32),
        pltpu.VMEM((_CH,), jnp.int32),
        pltpu.VMEM((_CH,), jnp.int32),
        pltpu.VMEM((_CH,), jnp.int32),
        pltpu.VMEM((_CH,), jnp.float32),
        pltpu.VMEM((_CH, _FW), jnp.float32),
        pltpu.VMEM((_CH, _FW), jnp.float32),
        pltpu.VMEM((_CH, _FW), jnp.float32),
        pltpu.VMEM((_CH, _FW), jnp.float32),
        pltpu.VMEM((_LN, _LN), jnp.float32),
        pltpu.SemaphoreType.DMA,
        pltpu.SemaphoreType.DMA,
        pltpu.SemaphoreType.DMA,
        pltpu.SemaphoreType.DMA,
    ],
)
def _gate_logit_sc(nodeA_hbm, eB_hbm, attl_hbm, xr_hbm, src_hbm, dst_hbm,
                   expv_hbm,
                   attl_v, xr_tab, src0, src1, dst0, dst1, exp_v,
                   rowsA0, rowsA1, rowsB0, rowsB1, accbuf,
                   semA0, semA1, semB0, semB1):
    c = lax.axis_index("c")
    s = lax.axis_index("s")
    wid = s * _NC + c
    pltpu.sync_copy(attl_hbm, attl_v)
    pltpu.sync_copy(xr_hbm, xr_tab)
    lanes = lax.iota(jnp.int32, _LN)
    bufs = ((src0, dst0, rowsA0, rowsB0, semA0, semB0),
            (src1, dst1, rowsA1, rowsB1, semA1, semB1))

    def start(i, b):
        srcb, dstb, rA, rB, sA, sB = bufs[b]
        base = wid * _EPT32 + i * _CH
        pltpu.sync_copy(src_hbm.at[pl.ds(base, _CH)], srcb)
        pltpu.sync_copy(dst_hbm.at[pl.ds(base, _CH)], dstb)
        pltpu.async_copy(nodeA_hbm.at[srcb], rA, sA)
        pltpu.async_copy(eB_hbm.at[pl.ds(base, _CH), :], rB, sB)

    def compute(i, b):
        srcb, dstb, rA, rB, sA, sB = bufs[b]
        base = wid * _EPT32 + i * _CH
        pltpu.make_async_copy(nodeA_hbm.at[srcb], rA, sA).wait()
        pltpu.make_async_copy(eB_hbm.at[pl.ds(base, _CH), :], rB, sB).wait()

        def grp(g, _):
            sl = pl.ds(g * _LN, _LN)

            def edge(j, _):
                row = g * _LN + j
                acc = jnp.zeros((_LN,), jnp.float32)
                for fc in range(_FW // _LN):
                    sl2 = pl.ds(fc * _LN, _LN)
                    v = rA[row, sl2] + rB[row, sl2]
                    v = jnp.maximum(v, NS * v)
                    acc = acc + v * attl_v[sl2]
                accbuf[j, :] = acc
                return 0
            lax.fori_loop(0, _LN, edge, 0)

            def tcol(jc, tot):
                colv = jnp.full((_LN,), jc, jnp.int32)
                return tot + plsc.load_gather(accbuf, [lanes, colv])
            tot = lax.fori_loop(0, _LN, tcol, jnp.zeros((_LN,), jnp.float32))
            lg = tot + plsc.load_gather(xr_tab, [dstb[sl]])
            lg = jnp.maximum(lg, NS * lg)
            exp_v[sl] = jnp.exp(lg)
            return 0

        lax.fori_loop(0, _CH // _LN, grp, 0)
        pltpu.sync_copy(exp_v, expv_hbm.at[pl.ds(base, _CH)])

    npairs = (_NCH32 - 1) // 2
    start(0, 0)

    def pair(p, _):
        i0 = 2 * p
        start(i0 + 1, 1)
        compute(i0, 0)

        @pl.when(p < npairs - 1)
        def _():
            start(i0 + 2, 0)
        compute(i0 + 1, 1)
        return 0

    lax.fori_loop(0, npairs, pair, 0)
    start(_NCH32 - 1, 0)
    compute(_NCH32 - 1, 0)


@functools.partial(
    pl.kernel,
    out_type=[jax.ShapeDtypeStruct((_NC, _NP, _HH), jnp.float32)],
    mesh=_sc_mesh,
    compiler_params=pltpu.CompilerParams(needs_layout_passes=False),
    scratch_types=[
        pltpu.VMEM((_CH,), jnp.int32),
        pltpu.VMEM((_CH,), jnp.int32),
        pltpu.VMEM((_CH,), jnp.int32),
        pltpu.VMEM((_CH,), jnp.int32),
        pltpu.VMEM((_CH,), jnp.float32),
        pltpu.VMEM((_CH,), jnp.float32),
        pltpu.VMEM((_CH, _HH), jnp.float32),
        pltpu.VMEM((_CH, _HH), jnp.float32),
        pltpu.VMEM_SHARED((_NP, _HH), jnp.float32),
        pltpu.SemaphoreType.DMA,
        pltpu.SemaphoreType.DMA,
    ],
)
def _wscatter_sc(src_hbm, dst_hbm, expv_hbm, table_hbm, zeros_hbm,
                 hsum_hbm,
                 srcg0, srcg1, dst0, dst1, exp0, exp1, rows0, rows1,
                 sh_h, semA0, semA1):
    c = lax.axis_index("c")
    s = lax.axis_index("s")
    nslice = pl.ds(s * _NPT, _NPT)
    pltpu.sync_copy(zeros_hbm.at[nslice, :], sh_h.at[nslice, :])
    plsc.subcore_barrier()
    cN = c * N
    bufs = ((srcg0, dst0, exp0, rows0, semA0),
            (srcg1, dst1, exp1, rows1, semA1))

    def start(i, b):
        srcgb, dstb, expb, rowsb, semb = bufs[b]
        base = s * _EPT + i * _CH
        pltpu.sync_copy(src_hbm.at[pl.ds(base, _CH)], srcgb)
        pltpu.sync_copy(dst_hbm.at[pl.ds(base, _CH)], dstb)
        pltpu.sync_copy(expv_hbm.at[pl.ds(base, _CH)], expb)

        def addoff(g, _):
            sl = pl.ds(g * _LN, _LN)
            srcgb[sl] = srcgb[sl] + cN
            return 0
        lax.fori_loop(0, _CH // _LN, addoff, 0)
        pltpu.async_copy(table_hbm.at[srcgb], rowsb, semb)

    def compute(b):
        srcgb, dstb, expb, rowsb, semb = bufs[b]
        pltpu.make_async_copy(table_hbm.at[srcgb], rowsb, semb).wait()

        def scale(g, _):
            ev = expb[pl.ds(g * _LN, _LN)]
            for j in range(_LN):
                wv = jnp.full((_LN,), ev[j], jnp.float32)
                row = g * _LN + j
                for k in range(_HH // _LN):
                    sl = pl.ds(k * _LN, _LN)
                    rowsb[row, sl] = rowsb[row, sl] * wv
            return 0
        lax.fori_loop(0, _CH // _LN, scale, 0)
        pltpu.sync_copy(rowsb, sh_h.at[dstb], add=True)

    start(0, 0)

    def pair(p, _):
        start(2 * p + 1, 1)
        compute(0)

        @pl.when(p < _NCHUNK // 2 - 1)
        def _():
            start(2 * p + 2, 0)
        compute(1)
        return 0

    lax.fori_loop(0, _NCHUNK // 2, pair, 0)
    plsc.subcore_barrier()
    pltpu.sync_copy(sh_h.at[nslice, :], hsum_hbm.at[c, nslice, :])


def _mk_table(xv):
    xp = jnp.pad(xv, ((0, 0), (0, _HPAD - H)))
    ones = jnp.ones((N, 1), jnp.float32)
    half1 = jnp.concatenate([xp[:, _HH:2 * _HH - 1], ones], axis=1)
    return jnp.concatenate([xp[:, :_HH], half1], axis=0)


def _norm_h(hsum):
    h = jnp.concatenate([hsum[0, :N], hsum[1, :N, :H - _HH]], axis=1)
    segsum = hsum[1, :N, _HH - 1]
    return h / (segsum[:, None] + 1e-16)


def _gate_edge(nodeA, eB, attl, xr, src, dst, xw2):
    nodeA = jnp.pad(nodeA, ((0, 0), (0, _FW - H)))
    eB = jnp.pad(eB, ((0, 0), (0, _FW - H)))
    attl = jnp.pad(attl, (0, _FW - H))
    xr = jnp.pad(xr, (0, _NP - N))
    (expv,) = _gate_logit_sc(nodeA, eB, attl, xr, src, dst)
    zeros = jnp.zeros((_NP, _HH), jnp.float32)
    (hsum,) = _wscatter_sc(src, dst, expv, _mk_table(xw2), zeros)
    return _norm_h(hsum)


def _lin1_body(x_ref, w_ref, b_ref, o_ref):
    acc = jnp.dot(x_ref[...], w_ref[...], preferred_element_type=jnp.float32)
    acc = acc + b_ref[...]
    o_ref[...] = jnp.where(acc > 0, acc, NS * acc)


@jax.jit
def _lin1(x, w, b):
    blk = 1000
    return pl.pallas_call(
        _lin1_body,
        grid=(N // blk,),
        in_specs=[
            pl.BlockSpec((blk, IN), lambda i: (i, 0)),
            pl.BlockSpec((IN, H), lambda i: (0, 0)),
            pl.BlockSpec((H,), lambda i: (0,)),
        ],
        out_specs=pl.BlockSpec((blk, H), lambda i: (i, 0)),
        out_shape=jax.ShapeDtypeStruct((N, H), jnp.float32),
    )(x, w, b)


def kernel(x, edge_index, edge_attr, batch, lin1_w, lin1_b, g_lin1_w, g_lin2_w, g_att_l, g_att_r, g_bias, gru0_wih, gru0_whh, gru0_bih, gru0_bhh, a_lin_w, a_att_src, a_att_dst, a_bias, gru1_wih, gru1_whh, gru1_bih, gru1_bhh, m_lin_w, m_att_src, m_att_dst, m_bias, mgru_wih, mgru_whh, mgru_bih, mgru_bhh, lin2_w, lin2_b):
    src, dst = edge_index[0], edge_index[1]
    x = _lin1(x, lin1_w, lin1_b)
    nodeA = x @ g_lin1_w[:H]
    eB = edge_attr @ g_lin1_w[H:]
    xr = x @ g_att_r
    h = _gate_edge(nodeA, eB, g_att_l, xr, src, dst, x @ g_lin2_w) + g_bias
    h = jax.nn.elu(h)
    x = jax.nn.relu(_gru(h, x, gru0_wih, gru0_whh, gru0_bih, gru0_bhh))
    xl = x @ a_lin_w
    h = _gat_edge(xl @ a_att_src, xl @ a_att_dst, src, dst, xl) + a_bias
    h = jax.nn.elu(h)
    x = jax.nn.relu(_gru(h, x, gru1_wih, gru1_whh, gru1_bih, gru1_bhh))
    onehot = (batch[None, :] == jnp.arange(B, dtype=batch.dtype)[:, None]).astype(jnp.float32)
    out = jax.nn.relu(onehot @ x)
    for _ in range(2):
        xs = x @ m_lin_w
        od = out @ m_lin_w
        alpha = _lrelu(xs @ m_att_src + onehot.T @ (od @ m_att_dst))
        e = jnp.exp(alpha)
        w = e / (onehot.T @ (onehot @ e) + 1e-16)
        h = onehot @ (xs * w[:, None]) + m_bias
        h = jax.nn.elu(h)
        out = jax.nn.relu(_gru(h, out, mgru_wih, mgru_whh, mgru_bih, mgru_bhh))
    return (out @ lin2_w + lin2_b).squeeze(-1)


# speedup vs baseline: 9.3254x; 1.0067x over previous
import functools

import jax
import jax.numpy as jnp
from jax import lax
from jax.experimental import pallas as pl
from jax.experimental.pallas import tpu as pltpu
from jax.experimental.pallas import tpu_sc as plsc

N = 10000
E = 320000
IN = 128
ED = 16
H = 200
B = 64
NS = 0.01


def _lrelu(v):
    return jnp.where(v > 0, v, NS * v)


def _seg_softmax(a, idx, num):
    m = jax.ops.segment_max(a, idx, num_segments=num)
    m = jnp.where(jnp.isfinite(m), m, 0.0)
    e = jnp.exp(a - m[idx])
    s = jax.ops.segment_sum(e, idx, num_segments=num)
    return e / (s[idx] + 1e-16)


def _gru(inp, h, wih, whh, bih, bhh):
    gi = inp @ wih.T + bih
    gh = h @ whh.T + bhh
    ir, iz, inn = jnp.split(gi, 3, axis=-1)
    hr, hz, hn = jnp.split(gh, 3, axis=-1)
    r = jax.nn.sigmoid(ir + hr)
    z = jax.nn.sigmoid(iz + hz)
    n = jnp.tanh(inn + r * hn)
    return (1.0 - z) * n + z * h


_NC, _NSUB, _LN = 2, 16, 16
_CH = 80
_EPT = E // _NSUB
_NCHUNK = _EPT // _CH
_NP = 10240
_NPT = _NP // _NSUB
_HH = 128
_HPAD = 2 * _HH

_sc_mesh = plsc.VectorSubcoreMesh(core_axis_name="c", subcore_axis_name="s")


@functools.partial(
    pl.kernel,
    out_type=[
        jax.ShapeDtypeStruct((_NC, _NP, _HH), jnp.float32),
    ],
    mesh=_sc_mesh,
    compiler_params=pltpu.CompilerParams(needs_layout_passes=False),
    scratch_types=[
        pltpu.VMEM((_NP,), jnp.float32),
        pltpu.VMEM((_NP,), jnp.float32),
        pltpu.VMEM((_CH,), jnp.int32),
        pltpu.VMEM((_CH,), jnp.int32),
        pltpu.VMEM((_CH,), jnp.int32),
        pltpu.VMEM((_CH,), jnp.int32),
        pltpu.VMEM((_CH,), jnp.float32),
        pltpu.VMEM((_CH,), jnp.float32),
        pltpu.VMEM((_CH, _HH), jnp.float32),
        pltpu.VMEM((_CH, _HH), jnp.float32),
        pltpu.VMEM_SHARED((_NP, _HH), jnp.float32),
        pltpu.SemaphoreType.DMA,
        pltpu.SemaphoreType.DMA,
    ],
)
def _gat_edge_sc(s_hbm, d_hbm, src_hbm, dst_hbm, table_hbm, zeros_hbm,
                 hsum_hbm,
                 s_tab, d_tab, srcg0, srcg1, dst0, dst1, exp0, exp1,
                 rows0, rows1, sh_h, semA0, semA1):
    c = lax.axis_index("c")
    s = lax.axis_index("s")
    nslice = pl.ds(s * _NPT, _NPT)
    pltpu.sync_copy(zeros_hbm.at[nslice, :], sh_h.at[nslice, :])
    pltpu.sync_copy(s_hbm, s_tab)
    pltpu.sync_copy(d_hbm, d_tab)
    plsc.subcore_barrier()
    cN = c * N
    bufs = ((srcg0, dst0, exp0, rows0, semA0),
            (srcg1, dst1, exp1, rows1, semA1))

    def start(i, b):
        srcgb, dstb, expb, rowsb, semb = bufs[b]
        base = s * _EPT + i * _CH
        pltpu.sync_copy(src_hbm.at[pl.ds(base, _CH)], srcgb)
        pltpu.sync_copy(dst_hbm.at[pl.ds(base, _CH)], dstb)

        def grp(g, _):
            sl = pl.ds(g * _LN, _LN)
            sv = srcgb[sl]
            dv = dstb[sl]
            srcgb[sl] = sv + cN
            a = plsc.load_gather(s_tab, [sv]) + plsc.load_gather(d_tab, [dv])
            a = jnp.maximum(a, NS * a)
            expb[sl] = jnp.exp(a)
            return 0
        lax.fori_loop(0, _CH // _LN, grp, 0)
        pltpu.async_copy(table_hbm.at[srcgb], rowsb, semb)

    def compute(b):
        srcgb, dstb, expb, rowsb, semb = bufs[b]
        pltpu.make_async_copy(table_hbm.at[srcgb], rowsb, semb).wait()

        def scale(g, _):
            ev = expb[pl.ds(g * _LN, _LN)]
            for j in range(_LN):
                wv = jnp.full((_LN,), ev[j], jnp.float32)
                row = g * _LN + j
                for k in range(_HH // _LN):
                    sl = pl.ds(k * _LN, _LN)
                    rowsb[row, sl] = rowsb[row, sl] * wv
            return 0
        lax.fori_loop(0, _CH // _LN, scale, 0)
        pltpu.sync_copy(rowsb, sh_h.at[dstb], add=True)

    start(0, 0)

    def pair(p, _):
        start(2 * p + 1, 1)
        compute(0)

        @pl.when(p < _NCHUNK // 2 - 1)
        def _():
            start(2 * p + 2, 0)
        compute(1)
        return 0

    lax.fori_loop(0, _NCHUNK // 2, pair, 0)
    plsc.subcore_barrier()
    pltpu.sync_copy(sh_h.at[nslice, :], hsum_hbm.at[c, nslice, :])


def _gat_edge(s, d, src, dst, xl):
    s = jnp.pad(s, (0, _NP - N))
    d = jnp.pad(d, (0, _NP - N))
    xlp = jnp.pad(xl, ((0, 0), (0, _HPAD - H)))
    ones = jnp.ones((N, 1), jnp.float32)
    half1 = jnp.concatenate([xlp[:, _HH:2 * _HH - 1], ones], axis=1)
    table = jnp.concatenate([xlp[:, :_HH], half1], axis=0)
    zeros = jnp.zeros((_NP, _HH), jnp.float32)
    (hsum,) = _gat_edge_sc(s, d, src, dst, table, zeros)
    h = jnp.concatenate([hsum[0, :N], hsum[1, :N, :H - _HH]], axis=1)
    segsum = hsum[1, :N, _HH - 1]
    return h / (segsum[:, None] + 1e-16)


_EPT32 = E // (_NC * _NSUB)
_NCH32 = _EPT32 // _CH
_FW = 2 * _HH


@functools.partial(
    pl.kernel,
    out_type=[jax.ShapeDtypeStruct((E,), jnp.float32)],
    mesh=_sc_mesh,
    compiler_params=pltpu.CompilerParams(needs_layout_passes=False),
    scratch_types=[
        pltpu.VMEM((_FW,), jnp.float32),
        pltpu.VMEM((_NP,), jnp.float32),
        pltpu.VMEM((_CH,), jnp.int32),
        pltpu.VMEM((_CH,), jnp.int32),
        pltpu.VMEM((_CH,), jnp.int32),
        pltpu.VMEM((_CH,), jnp.int32),
        pltpu.VMEM((_CH,), jnp.float32),
        pltpu.VMEM((_CH, _FW), jnp.float32),
        pltpu.VMEM((_CH, _FW), jnp.float32),
        pltpu.VMEM((_CH, _FW), jnp.float32),
        pltpu.VMEM((_CH, _FW), jnp.float32),
        pltpu.VMEM((_LN, _LN), jnp.float32),
        pltpu.SemaphoreType.DMA,
        pltpu.SemaphoreType.DMA,
        pltpu.SemaphoreType.DMA,
        pltpu.SemaphoreType.DMA,
    ],
)
def _gate_logit_sc(nodeA_hbm, eB_hbm, attl_hbm, xr_hbm, src_hbm, dst_hbm,
                   expv_hbm,
                   attl_v, xr_tab, src0, src1, dst0, dst1, exp_v,
                   rowsA0, rowsA1, rowsB0, rowsB1, accbuf,
                   semA0, semA1, semB0, semB1):
    c = lax.axis_index("c")
    s = lax.axis_index("s")
    wid = s * _NC + c
    pltpu.sync_copy(attl_hbm, attl_v)
    pltpu.sync_copy(xr_hbm, xr_tab)
    lanes = lax.iota(jnp.int32, _LN)
    bufs = ((src0, dst0, rowsA0, rowsB0, semA0, semB0),
            (src1, dst1, rowsA1, rowsB1, semA1, semB1))

    def start(i, b):
        srcb, dstb, rA, rB, sA, sB = bufs[b]
        base = wid * _EPT32 + i * _CH
        pltpu.sync_copy(src_hbm.at[pl.ds(base, _CH)], srcb)
        pltpu.sync_copy(dst_hbm.at[pl.ds(base, _CH)], dstb)
        pltpu.async_copy(nodeA_hbm.at[srcb], rA, sA)
        pltpu.async_copy(eB_hbm.at[pl.ds(base, _CH), :], rB, sB)

    def compute(i, b):
        srcb, dstb, rA, rB, sA, sB = bufs[b]
        base = wid * _EPT32 + i * _CH
        pltpu.make_async_copy(nodeA_hbm.at[srcb], rA, sA).wait()
        pltpu.make_async_copy(eB_hbm.at[pl.ds(base, _CH), :], rB, sB).wait()

        def grp(g, _):
            sl = pl.ds(g * _LN, _LN)

            def edge(j, _):
                row = g * _LN + j
                acc = jnp.zeros((_LN,), jnp.float32)
                for fc in range(_FW // _LN):
                    sl2 = pl.ds(fc * _LN, _LN)
                    v = rA[row, sl2] + rB[row, sl2]
                    v = jnp.maximum(v, NS * v)
                    acc = acc + v * attl_v[sl2]
                accbuf[j, :] = acc
                return 0
            lax.fori_loop(0, _LN, edge, 0)

            def tcol(jc, tot):
                colv = jnp.full((_LN,), jc, jnp.int32)
                return tot + plsc.load_gather(accbuf, [lanes, colv])
            tot = lax.fori_loop(0, _LN, tcol, jnp.zeros((_LN,), jnp.float32))
            lg = tot + plsc.load_gather(xr_tab, [dstb[sl]])
            lg = jnp.maximum(lg, NS * lg)
            exp_v[sl] = jnp.exp(lg)
            return 0

        lax.fori_loop(0, _CH // _LN, grp, 0)
        pltpu.sync_copy(exp_v, expv_hbm.at[pl.ds(base, _CH)])

    npairs = (_NCH32 - 1) // 2
    start(0, 0)

    def pair(p, _):
        i0 = 2 * p
        start(i0 + 1, 1)
        compute(i0, 0)

        @pl.when(p < npairs - 1)
        def _():
            start(i0 + 2, 0)
        compute(i0 + 1, 1)
        return 0

    lax.fori_loop(0, npairs, pair, 0)
    start(_NCH32 - 1, 0)
    compute(_NCH32 - 1, 0)


@functools.partial(
    pl.kernel,
    out_type=[jax.ShapeDtypeStruct((_NC, _NP, _HH), jnp.float32)],
    mesh=_sc_mesh,
    compiler_params=pltpu.CompilerParams(needs_layout_passes=False),
    scratch_types=[
        pltpu.VMEM((_CH,), jnp.int32),
        pltpu.VMEM((_CH,), jnp.int32),
        pltpu.VMEM((_CH,), jnp.int32),
        pltpu.VMEM((_CH,), jnp.int32),
        pltpu.VMEM((_CH,), jnp.float32),
        pltpu.VMEM((_CH,), jnp.float32),
        pltpu.VMEM((_CH, _HH), jnp.float32),
        pltpu.VMEM((_CH, _HH), jnp.float32),
        pltpu.VMEM_SHARED((_NP, _HH), jnp.float32),
        pltpu.SemaphoreType.DMA,
        pltpu.SemaphoreType.DMA,
    ],
)
def _wscatter_sc(src_hbm, dst_hbm, expv_hbm, table_hbm, zeros_hbm,
                 hsum_hbm,
                 srcg0, srcg1, dst0, dst1, exp0, exp1, rows0, rows1,
                 sh_h, semA0, semA1):
    c = lax.axis_index("c")
    s = lax.axis_index("s")
    nslice = pl.ds(s * _NPT, _NPT)
    pltpu.sync_copy(zeros_hbm.at[nslice, :], sh_h.at[nslice, :])
    plsc.subcore_barrier()
    cN = c * N
    bufs = ((srcg0, dst0, exp0, rows0, semA0),
            (srcg1, dst1, exp1, rows1, semA1))

    def start(i, b):
        srcgb, dstb, expb, rowsb, semb = bufs[b]
        base = s * _EPT + i * _CH
        pltpu.sync_copy(src_hbm.at[pl.ds(base, _CH)], srcgb)
        pltpu.sync_copy(dst_hbm.at[pl.ds(base, _CH)], dstb)
        pltpu.sync_copy(expv_hbm.at[pl.ds(base, _CH)], expb)

        def addoff(g, _):
            sl = pl.ds(g * _LN, _LN)
            srcgb[sl] = srcgb[sl] + cN
            return 0
        lax.fori_loop(0, _CH // _LN, addoff, 0)
        pltpu.async_copy(table_hbm.at[srcgb], rowsb, semb)

    def compute(b):
        srcgb, dstb, expb, rowsb, semb = bufs[b]
        pltpu.make_async_copy(table_hbm.at[srcgb], rowsb, semb).wait()

        def scale(g, _):
            ev = expb[pl.ds(g * _LN, _LN)]
            for j in range(_LN):
                wv = jnp.full((_LN,), ev[j], jnp.float32)
                row = g * _LN + j
                for k in range(_HH // _LN):
                    sl = pl.ds(k * _LN, _LN)
                    rowsb[row, sl] = rowsb[row, sl] * wv
            return 0
        lax.fori_loop(0, _CH // _LN, scale, 0)
        pltpu.sync_copy(rowsb, sh_h.at[dstb], add=True)

    start(0, 0)

    def pair(p, _):
        start(2 * p + 1, 1)
        compute(0)

        @pl.when(p < _NCHUNK // 2 - 1)
        def _():
            start(2 * p + 2, 0)
        compute(1)
        return 0

    lax.fori_loop(0, _NCHUNK // 2, pair, 0)
    plsc.subcore_barrier()
    pltpu.sync_copy(sh_h.at[nslice, :], hsum_hbm.at[c, nslice, :])


def _mk_table(xv):
    xp = jnp.pad(xv, ((0, 0), (0, _HPAD - H)))
    ones = jnp.ones((N, 1), jnp.float32)
    half1 = jnp.concatenate([xp[:, _HH:2 * _HH - 1], ones], axis=1)
    return jnp.concatenate([xp[:, :_HH], half1], axis=0)


def _norm_h(hsum):
    h = jnp.concatenate([hsum[0, :N], hsum[1, :N, :H - _HH]], axis=1)
    segsum = hsum[1, :N, _HH - 1]
    return h / (segsum[:, None] + 1e-16)


def _gate_edge(nodeA, eB, attl, xr, src, dst, xw2):
    nodeA = jnp.pad(nodeA, ((0, 0), (0, _FW - H)))
    eB = jnp.pad(eB, ((0, 0), (0, _FW - H)))
    attl = jnp.pad(attl, (0, _FW - H))
    xr = jnp.pad(xr, (0, _NP - N))
    (expv,) = _gate_logit_sc(nodeA, eB, attl, xr, src, dst)
    zeros = jnp.zeros((_NP, _HH), jnp.float32)
    (hsum,) = _wscatter_sc(src, dst, expv, _mk_table(xw2), zeros)
    return _norm_h(hsum)


def _gru_body(hp_ref, x_ref, bias_ref, wiht_ref, whht_ref, bih_ref, bhh_ref, o_ref):
    hp = hp_ref[...] + bias_ref[...]
    hin = jnp.where(hp > 0, hp, jnp.exp(jnp.minimum(hp, 0.0)) - 1.0)
    xv = x_ref[...]
    gi = jnp.dot(hin, wiht_ref[...], preferred_element_type=jnp.float32) + bih_ref[...]
    gh = jnp.dot(xv, whht_ref[...], preferred_element_type=jnp.float32) + bhh_ref[...]
    ir, iz, inn = gi[:, :H], gi[:, H:2 * H], gi[:, 2 * H:]
    hr, hz, hn = gh[:, :H], gh[:, H:2 * H], gh[:, 2 * H:]
    r = jax.nn.sigmoid(ir + hr)
    z = jax.nn.sigmoid(iz + hz)
    n = jnp.tanh(inn + r * hn)
    o_ref[...] = jnp.maximum((1.0 - z) * n + z * xv, 0.0)


def _node_gru(h_pre, x, bias, wih, whh, bih, bhh):
    blk = 1000
    return pl.pallas_call(
        _gru_body,
        grid=(N // blk,),
        in_specs=[
            pl.BlockSpec((blk, H), lambda i: (i, 0)),
            pl.BlockSpec((blk, H), lambda i: (i, 0)),
            pl.BlockSpec((H,), lambda i: (0,)),
            pl.BlockSpec((H, 3 * H), lambda i: (0, 0)),
            pl.BlockSpec((H, 3 * H), lambda i: (0, 0)),
            pl.BlockSpec((3 * H,), lambda i: (0,)),
            pl.BlockSpec((3 * H,), lambda i: (0,)),
        ],
        out_specs=pl.BlockSpec((blk, H), lambda i: (i, 0)),
        out_shape=jax.ShapeDtypeStruct((N, H), jnp.float32),
    )(h_pre, x, bias, wih.T, whh.T, bih, bhh)


def _lin1_body(x_ref, w_ref, b_ref, o_ref):
    acc = jnp.dot(x_ref[...], w_ref[...], preferred_element_type=jnp.float32)
    acc = acc + b_ref[...]
    o_ref[...] = jnp.where(acc > 0, acc, NS * acc)


@jax.jit
def _lin1(x, w, b):
    blk = 1000
    return pl.pallas_call(
        _lin1_body,
        grid=(N // blk,),
        in_specs=[
            pl.BlockSpec((blk, IN), lambda i: (i, 0)),
            pl.BlockSpec((IN, H), lambda i: (0, 0)),
            pl.BlockSpec((H,), lambda i: (0,)),
        ],
        out_specs=pl.BlockSpec((blk, H), lambda i: (i, 0)),
        out_shape=jax.ShapeDtypeStruct((N, H), jnp.float32),
    )(x, w, b)


def kernel(x, edge_index, edge_attr, batch, lin1_w, lin1_b, g_lin1_w, g_lin2_w, g_att_l, g_att_r, g_bias, gru0_wih, gru0_whh, gru0_bih, gru0_bhh, a_lin_w, a_att_src, a_att_dst, a_bias, gru1_wih, gru1_whh, gru1_bih, gru1_bhh, m_lin_w, m_att_src, m_att_dst, m_bias, mgru_wih, mgru_whh, mgru_bih, mgru_bhh, lin2_w, lin2_b):
    src, dst = edge_index[0], edge_index[1]
    x = _lin1(x, lin1_w, lin1_b)
    nodeA = x @ g_lin1_w[:H]
    eB = edge_attr @ g_lin1_w[H:]
    xr = x @ g_att_r
    h = _gate_edge(nodeA, eB, g_att_l, xr, src, dst, x @ g_lin2_w)
    x = _node_gru(h, x, g_bias, gru0_wih, gru0_whh, gru0_bih, gru0_bhh)
    xl = x @ a_lin_w
    h = _gat_edge(xl @ a_att_src, xl @ a_att_dst, src, dst, xl)
    x = _node_gru(h, x, a_bias, gru1_wih, gru1_whh, gru1_bih, gru1_bhh)
    onehot = (batch[None, :] == jnp.arange(B, dtype=batch.dtype)[:, None]).astype(jnp.float32)
    out = jax.nn.relu(onehot @ x)
    for _ in range(2):
        xs = x @ m_lin_w
        od = out @ m_lin_w
        alpha = _lrelu(xs @ m_att_src + onehot.T @ (od @ m_att_dst))
        e = jnp.exp(alpha)
        w = e / (onehot.T @ (onehot @ e) + 1e-16)
        h = onehot @ (xs * w[:, None]) + m_bias
        h = jax.nn.elu(h)
        out = jax.nn.relu(_gru(h, out, mgru_wih, mgru_whh, mgru_bih, mgru_bhh))
    return (out @ lin2_w + lin2_b).squeeze(-1)
